# Initial kernel scaffold; baseline (speedup 1.0000x reference)
import jax
import jax.numpy as jnp
from jax.experimental import pallas as pl


def kernel(nodes, edges, receivers, senders, W_node, W_edge):
    raise NotImplementedError("write your pallas kernel here")


# speedup vs baseline: 33.3768x; 33.3768x over previous
import functools

import jax
import jax.numpy as jnp
from jax import lax
from jax.experimental import pallas as pl
from jax.experimental.pallas import tpu as pltpu
from jax.experimental.pallas import tpu_sc as plsc

NC = 2
NS = 16
NW = NC * NS
L = 16

DN = 128
DE = 16
N = 10000
E = 320000
K = E - N

MESH = dict(
    mesh=plsc.VectorSubcoreMesh(
        core_axis_name="c", subcore_axis_name="s", num_cores=NC, num_subcores=NS),
    compiler_params=pltpu.CompilerParams(needs_layout_passes=False),
)


def _wid():
    return lax.axis_index("s") * NC + lax.axis_index("c")


_CHM = E // NW


@functools.partial(
    pl.kernel,
    out_type=jax.ShapeDtypeStruct((NW * L,), jnp.int32),
    scratch_types=[
        pltpu.VMEM((_CHM,), jnp.int32),
        pltpu.VMEM((_CHM,), jnp.int32),
        pltpu.VMEM((L,), jnp.int32),
    ],
    **MESH,
)
def _count_kernel(s_hbm, r_hbm, out_hbm, sbuf, rbuf, cbuf):
    w = _wid()
    base = w * _CHM
    pltpu.sync_copy(s_hbm.at[pl.ds(base, _CHM)], sbuf)
    pltpu.sync_copy(r_hbm.at[pl.ds(base, _CHM)], rbuf)

    def body(j, cnt):
        sv = sbuf[pl.ds(j * L, L)]
        rv = rbuf[pl.ds(j * L, L)]
        return cnt + jnp.where(sv != rv, 1, 0).astype(jnp.int32)

    cnt = lax.fori_loop(0, _CHM // L, body, jnp.zeros((L,), jnp.int32))
    cbuf[...] = jnp.broadcast_to(jnp.sum(cnt), (L,))
    pltpu.sync_copy(cbuf, out_hbm.at[pl.ds(w * L, L)])


@functools.partial(
    pl.kernel,
    out_type=(
        jax.ShapeDtypeStruct((E,), jnp.int32),
        jax.ShapeDtypeStruct((E,), jnp.int32),
    ),
    scratch_types=[
        pltpu.VMEM((_CHM,), jnp.int32),
        pltpu.VMEM((_CHM,), jnp.int32),
        pltpu.VMEM((_CHM,), jnp.int32),
        pltpu.VMEM((_CHM,), jnp.int32),
        pltpu.VMEM((NW * L,), jnp.int32),
    ],
    **MESH,
)
def _rewrite_kernel(s_hbm, r_hbm, cnt_hbm, s2_hbm, r2_hbm, sbuf, rbuf, s2buf, r2buf, cntbuf):
    w = _wid()
    base = w * _CHM
    pltpu.sync_copy(s_hbm.at[pl.ds(base, _CHM)], sbuf)
    pltpu.sync_copy(r_hbm.at[pl.ds(base, _CHM)], rbuf)
    pltpu.sync_copy(cnt_hbm, cntbuf)

    off = jnp.zeros((L,), jnp.int32)
    tot = jnp.zeros((L,), jnp.int32)
    for k in range(NW):
        ck = cntbuf[pl.ds(k * L, L)]
        off = off + jnp.where(jnp.broadcast_to(k < w, (L,)), ck, 0)
        tot = tot + ck
    lane = lax.iota(jnp.int32, L)
    pad_force = jnp.broadcast_to((w == 0) & (jnp.sum(tot) // L < K), (L,)) & (lane == 0)

    def body(j, running):
        sv = sbuf[pl.ds(j * L, L)]
        rv = rbuf[pl.ds(j * L, L)]
        m = sv != rv
        mi = jnp.where(m, 1, 0).astype(jnp.int32)
        incl = plsc.cumsum(mi)
        rank = incl + running
        upd = m & (rank <= K)
        upd = upd | (pad_force & jnp.broadcast_to(j == 0, (L,)))
        s2buf[pl.ds(j * L, L)] = jnp.where(upd, sv, N)
        r2buf[pl.ds(j * L, L)] = jnp.where(upd, rv, N)
        return running + jnp.max(incl)

    lax.fori_loop(0, _CHM // L, body, jnp.sum(off) // L)
    pltpu.sync_copy(s2buf, s2_hbm.at[pl.ds(base, _CHM)])
    pltpu.sync_copy(r2buf, r2_hbm.at[pl.ds(base, _CHM)])


_CH1 = 16000


@functools.partial(
    pl.kernel,
    out_type=jax.ShapeDtypeStruct((2 * DE * N,), jnp.float32),
    scratch_types=[
        pltpu.VMEM((N,), jnp.float32),
        pltpu.VMEM((_CH1,), jnp.float32),
        pltpu.VMEM((_CH1,), jnp.int32),
    ],
    **MESH,
)
def _segsum_kernel(edges_hbm, sridx_hbm, out_hbm, acc, vbuf, ibuf):
    w = _wid()
    t = w // DE
    f = w % DE

    def zbody(i, _):
        acc[pl.ds(i * L, L)] = jnp.zeros((L,), jnp.float32)
        return 0

    lax.fori_loop(0, N // L, zbody, 0)

    def chunk(c, _):
        base = c * _CH1
        pltpu.sync_copy(edges_hbm.at[pl.ds(f * E + base, _CH1)], vbuf)
        pltpu.sync_copy(sridx_hbm.at[pl.ds(t * E + base, _CH1)], ibuf)

        def body(j, _):
            iv = ibuf[pl.ds(j * L, L)]
            vv = vbuf[pl.ds(j * L, L)]
            plsc.addupdate_scatter(acc, [iv], vv)
            return 0

        lax.fori_loop(0, _CH1 // L, body, 0)
        return 0

    lax.fori_loop(0, E // _CH1, chunk, 0)
    pltpu.sync_copy(acc, out_hbm.at[pl.ds(w * N, N)])


_CH2 = 16000
_HALF = E // 2


@functools.partial(
    pl.kernel,
    out_type=jax.ShapeDtypeStruct((DE * E,), jnp.float32),
    scratch_types=[
        pltpu.VMEM((N + L,), jnp.float32),
        pltpu.VMEM((N + L,), jnp.float32),
        pltpu.VMEM((_CH2,), jnp.float32),
        pltpu.VMEM((_CH2,), jnp.int32),
        pltpu.VMEM((_CH2,), jnp.int32),
        pltpu.VMEM((_CH2,), jnp.float32),
    ],
    **MESH,
)
def _edge_kernel(c2_hbm, ab_hbm, s2_hbm, r2_hbm, out_hbm, arow, brow, cbuf, sbuf, rbuf, obuf):
    w = _wid()
    h = w // DE
    f = w % DE
    pltpu.sync_copy(ab_hbm.at[pl.ds(f * N, N)], arow.at[pl.ds(0, N)])
    pltpu.sync_copy(ab_hbm.at[pl.ds((DE + f) * N, N)], brow.at[pl.ds(0, N)])
    arow[pl.ds(N, L)] = jnp.zeros((L,), jnp.float32)
    brow[pl.ds(N, L)] = jnp.zeros((L,), jnp.float32)

    def chunk(c, _):
        base = h * _HALF + c * _CH2
        pltpu.sync_copy(c2_hbm.at[pl.ds(f * E + base, _CH2)], cbuf)
        pltpu.sync_copy(s2_hbm.at[pl.ds(base, _CH2)], sbuf)
        pltpu.sync_copy(r2_hbm.at[pl.ds(base, _CH2)], rbuf)

        def body(j, _):
            sv = sbuf[pl.ds(j * L, L)]
            rv = rbuf[pl.ds(j * L, L)]
            cv = cbuf[pl.ds(j * L, L)]
            av = plsc.load_gather(arow, [sv])
            bv = plsc.load_gather(brow, [rv])
            obuf[pl.ds(j * L, L)] = cv + av + bv
            return 0

        lax.fori_loop(0, _CH2 // L, body, 0)
        pltpu.sync_copy(obuf, out_hbm.at[pl.ds(f * E + base, _CH2)])
        return 0

    lax.fori_loop(0, _HALF // _CH2, chunk, 0)


def _node_body(nodes_ref, sr_ref, wnn_ref, wnsr_ref, wsr_ref, nodes_out, ab_out):
    dot = functools.partial(
        jnp.dot, precision=lax.Precision.HIGHEST, preferred_element_type=jnp.float32
    )
    nn = dot(wnn_ref[...], nodes_ref[...]) + dot(wnsr_ref[...], sr_ref[...])
    nodes_out[...] = nn
    ab_out[...] = dot(wsr_ref[...], nn)


def _node_update(nodes, SR, Wn_n, Wn_sr, W_sr):
    return pl.pallas_call(
        _node_body,
        out_shape=(
            jax.ShapeDtypeStruct((DN, N), jnp.float32),
            jax.ShapeDtypeStruct((2 * DE, N), jnp.float32),
        ),
    )(nodes, SR, Wn_n, Wn_sr, W_sr)


_BC = 16000


def _c_body(wee_ref, e_ref, s2_ref, out_ref):
    c = jnp.dot(
        wee_ref[...], e_ref[...],
        precision=lax.Precision.HIGHEST, preferred_element_type=jnp.float32,
    )
    out_ref[...] = jnp.where(s2_ref[...] != N, c, e_ref[...])


def _c_update(W_ee, edges, s2_2d):
    return pl.pallas_call(
        _c_body,
        grid=(E // _BC,),
        in_specs=[
            pl.BlockSpec((DE, DE), lambda i: (0, 0)),
            pl.BlockSpec((DE, _BC), lambda i: (0, i)),
            pl.BlockSpec((1, _BC), lambda i: (0, i)),
        ],
        out_specs=pl.BlockSpec((DE, _BC), lambda i: (0, i)),
        out_shape=jax.ShapeDtypeStruct((DE, E), jnp.float32),
    )(W_ee, edges, s2_2d)


def kernel(nodes, edges, receivers, senders, W_node, W_edge):
    Wn_n = W_node[:, :DN]
    Wn_sr = W_node[:, DN:DN + 2 * DE]
    W_ee = W_edge[:, :DE]
    W_sr = jnp.concatenate([W_edge[:, DE:DE + DN], W_edge[:, DE + DN:]], axis=0)
    sr_idx = jnp.concatenate([senders, receivers])

    counts = _count_kernel(senders, receivers)
    s2, r2 = _rewrite_kernel(senders, receivers, counts)
    s2_2d = s2.reshape(1, E)

    edges_flat = edges.reshape(-1)
    for _ in range(2):
        SR = _segsum_kernel(edges_flat, sr_idx).reshape(2 * DE, N)
        nodes, AB = _node_update(nodes, SR, Wn_n, Wn_sr, W_sr)
        C2 = _c_update(W_ee, edges, s2_2d)
        edges_flat = _edge_kernel(C2.reshape(-1), AB.reshape(-1), s2, r2)
        edges = edges_flat.reshape(DE, E)
    return nodes, edges, receivers, senders


# speedup vs baseline: 46.2201x; 1.3848x over previous
import functools

import jax
import jax.numpy as jnp
from jax import lax
from jax.experimental import pallas as pl
from jax.experimental.pallas import tpu as pltpu
from jax.experimental.pallas import tpu_sc as plsc

NC = 2
NS = 16
NW = NC * NS
L = 16

DN = 128
DE = 16
N = 10000
E = 320000
K = E - N

MESH = dict(
    mesh=plsc.VectorSubcoreMesh(
        core_axis_name="c", subcore_axis_name="s", num_cores=NC, num_subcores=NS),
    compiler_params=pltpu.CompilerParams(needs_layout_passes=False),
)


def _wid():
    return lax.axis_index("s") * NC + lax.axis_index("c")


_CHM = E // NW


@functools.partial(
    pl.kernel,
    out_type=jax.ShapeDtypeStruct((NW * L,), jnp.int32),
    scratch_types=[
        pltpu.VMEM((_CHM,), jnp.int32),
        pltpu.VMEM((_CHM,), jnp.int32),
        pltpu.VMEM((L,), jnp.int32),
    ],
    **MESH,
)
def _count_kernel(s_hbm, r_hbm, out_hbm, sbuf, rbuf, cbuf):
    w = _wid()
    base = w * _CHM
    pltpu.sync_copy(s_hbm.at[pl.ds(base, _CHM)], sbuf)
    pltpu.sync_copy(r_hbm.at[pl.ds(base, _CHM)], rbuf)

    def body(j, cnt):
        sv = sbuf[pl.ds(j * L, L)]
        rv = rbuf[pl.ds(j * L, L)]
        return cnt + jnp.where(sv != rv, 1, 0).astype(jnp.int32)

    cnt = lax.fori_loop(0, _CHM // L, body, jnp.zeros((L,), jnp.int32))
    cbuf[...] = jnp.broadcast_to(jnp.sum(cnt), (L,))
    pltpu.sync_copy(cbuf, out_hbm.at[pl.ds(w * L, L)])


@functools.partial(
    pl.kernel,
    out_type=(
        jax.ShapeDtypeStruct((E,), jnp.int32),
        jax.ShapeDtypeStruct((E,), jnp.int32),
    ),
    scratch_types=[
        pltpu.VMEM((_CHM,), jnp.int32),
        pltpu.VMEM((_CHM,), jnp.int32),
        pltpu.VMEM((_CHM,), jnp.int32),
        pltpu.VMEM((_CHM,), jnp.int32),
        pltpu.VMEM((NW * L,), jnp.int32),
    ],
    **MESH,
)
def _rewrite_kernel(s_hbm, r_hbm, cnt_hbm, s2_hbm, r2_hbm, sbuf, rbuf, s2buf, r2buf, cntbuf):
    w = _wid()
    base = w * _CHM
    pltpu.sync_copy(s_hbm.at[pl.ds(base, _CHM)], sbuf)
    pltpu.sync_copy(r_hbm.at[pl.ds(base, _CHM)], rbuf)
    pltpu.sync_copy(cnt_hbm, cntbuf)

    off = jnp.zeros((L,), jnp.int32)
    tot = jnp.zeros((L,), jnp.int32)
    for k in range(NW):
        ck = cntbuf[pl.ds(k * L, L)]
        off = off + jnp.where(jnp.broadcast_to(k < w, (L,)), ck, 0)
        tot = tot + ck
    lane = lax.iota(jnp.int32, L)
    pad_force = jnp.broadcast_to((w == 0) & (jnp.sum(tot) // L < K), (L,)) & (lane == 0)

    def body(j, running):
        sv = sbuf[pl.ds(j * L, L)]
        rv = rbuf[pl.ds(j * L, L)]
        m = sv != rv
        mi = jnp.where(m, 1, 0).astype(jnp.int32)
        incl = plsc.cumsum(mi)
        rank = incl + running
        upd = m & (rank <= K)
        upd = upd | (pad_force & jnp.broadcast_to(j == 0, (L,)))
        s2buf[pl.ds(j * L, L)] = jnp.where(upd, sv, N)
        r2buf[pl.ds(j * L, L)] = jnp.where(upd, rv, N)
        return running + jnp.max(incl)

    lax.fori_loop(0, _CHM // L, body, jnp.sum(off) // L)
    pltpu.sync_copy(s2buf, s2_hbm.at[pl.ds(base, _CHM)])
    pltpu.sync_copy(r2buf, r2_hbm.at[pl.ds(base, _CHM)])


_CH1 = 16000


@functools.partial(
    pl.kernel,
    out_type=jax.ShapeDtypeStruct((2 * DE * N,), jnp.float32),
    scratch_types=[
        pltpu.VMEM((N,), jnp.float32),
        pltpu.VMEM((_CH1,), jnp.float32),
        pltpu.VMEM((_CH1,), jnp.int32),
    ],
    **MESH,
)
def _segsum_kernel(edges_hbm, sridx_hbm, out_hbm, acc, vbuf, ibuf):
    w = _wid()
    t = w // DE
    f = w % DE

    @plsc.parallel_loop(0, N, L, unroll=8)
    def _zero(i):
        acc[pl.ds(i, L)] = jnp.zeros((L,), jnp.float32)

    def chunk(c, _):
        base = c * _CH1
        pltpu.sync_copy(edges_hbm.at[pl.ds(f * E + base, _CH1)], vbuf)
        pltpu.sync_copy(sridx_hbm.at[pl.ds(t * E + base, _CH1)], ibuf)

        @plsc.parallel_loop(0, _CH1, L, unroll=8)
        def _scat(j):
            iv = ibuf[pl.ds(j, L)]
            vv = vbuf[pl.ds(j, L)]
            plsc.addupdate_scatter(acc, [iv], vv)

        return 0

    lax.fori_loop(0, E // _CH1, chunk, 0)
    pltpu.sync_copy(acc, out_hbm.at[pl.ds(w * N, N)])


_CH2 = 16000
_HALF = E // 2


@functools.partial(
    pl.kernel,
    out_type=jax.ShapeDtypeStruct((DE * E,), jnp.float32),
    scratch_types=[
        pltpu.VMEM((N + L,), jnp.float32),
        pltpu.VMEM((N + L,), jnp.float32),
        pltpu.VMEM((_CH2,), jnp.float32),
        pltpu.VMEM((_CH2,), jnp.int32),
        pltpu.VMEM((_CH2,), jnp.int32),
        pltpu.VMEM((_CH2,), jnp.float32),
    ],
    **MESH,
)
def _edge_kernel(c2_hbm, ab_hbm, s2_hbm, r2_hbm, out_hbm, arow, brow, cbuf, sbuf, rbuf, obuf):
    w = _wid()
    h = w // DE
    f = w % DE
    pltpu.sync_copy(ab_hbm.at[pl.ds(f * N, N)], arow.at[pl.ds(0, N)])
    pltpu.sync_copy(ab_hbm.at[pl.ds((DE + f) * N, N)], brow.at[pl.ds(0, N)])
    arow[pl.ds(N, L)] = jnp.zeros((L,), jnp.float32)
    brow[pl.ds(N, L)] = jnp.zeros((L,), jnp.float32)

    def chunk(c, _):
        base = h * _HALF + c * _CH2
        pltpu.sync_copy(c2_hbm.at[pl.ds(f * E + base, _CH2)], cbuf)
        pltpu.sync_copy(s2_hbm.at[pl.ds(base, _CH2)], sbuf)
        pltpu.sync_copy(r2_hbm.at[pl.ds(base, _CH2)], rbuf)

        @plsc.parallel_loop(0, _CH2, L, unroll=8)
        def _gat(j):
            sv = sbuf[pl.ds(j, L)]
            rv = rbuf[pl.ds(j, L)]
            cv = cbuf[pl.ds(j, L)]
            av = plsc.load_gather(arow, [sv])
            bv = plsc.load_gather(brow, [rv])
            obuf[pl.ds(j, L)] = cv + av + bv
        pltpu.sync_copy(obuf, out_hbm.at[pl.ds(f * E + base, _CH2)])
        return 0

    lax.fori_loop(0, _HALF // _CH2, chunk, 0)


def _node_body(nodes_ref, sr_ref, wnn_ref, wnsr_ref, wsr_ref, nodes_out, ab_out):
    dot = functools.partial(
        jnp.dot, precision=lax.Precision.HIGHEST, preferred_element_type=jnp.float32
    )
    nn = dot(wnn_ref[...], nodes_ref[...]) + dot(wnsr_ref[...], sr_ref[...])
    nodes_out[...] = nn
    ab_out[...] = dot(wsr_ref[...], nn)


def _node_update(nodes, SR, Wn_n, Wn_sr, W_sr):
    return pl.pallas_call(
        _node_body,
        out_shape=(
            jax.ShapeDtypeStruct((DN, N), jnp.float32),
            jax.ShapeDtypeStruct((2 * DE, N), jnp.float32),
        ),
    )(nodes, SR, Wn_n, Wn_sr, W_sr)


_BC = 16000


def _c_body(wee_ref, e_ref, s2_ref, out_ref):
    c = jnp.dot(
        wee_ref[...], e_ref[...],
        precision=lax.Precision.HIGHEST, preferred_element_type=jnp.float32,
    )
    out_ref[...] = jnp.where(s2_ref[...] != N, c, e_ref[...])


def _c_update(W_ee, edges, s2_2d):
    return pl.pallas_call(
        _c_body,
        grid=(E // _BC,),
        in_specs=[
            pl.BlockSpec((DE, DE), lambda i: (0, 0)),
            pl.BlockSpec((DE, _BC), lambda i: (0, i)),
            pl.BlockSpec((1, _BC), lambda i: (0, i)),
        ],
        out_specs=pl.BlockSpec((DE, _BC), lambda i: (0, i)),
        out_shape=jax.ShapeDtypeStruct((DE, E), jnp.float32),
    )(W_ee, edges, s2_2d)


def kernel(nodes, edges, receivers, senders, W_node, W_edge):
    Wn_n = W_node[:, :DN]
    Wn_sr = W_node[:, DN:DN + 2 * DE]
    W_ee = W_edge[:, :DE]
    W_sr = jnp.concatenate([W_edge[:, DE:DE + DN], W_edge[:, DE + DN:]], axis=0)
    sr_idx = jnp.concatenate([senders, receivers])

    counts = _count_kernel(senders, receivers)
    s2, r2 = _rewrite_kernel(senders, receivers, counts)
    s2_2d = s2.reshape(1, E)

    edges_flat = edges.reshape(-1)
    for _ in range(2):
        SR = _segsum_kernel(edges_flat, sr_idx).reshape(2 * DE, N)
        nodes, AB = _node_update(nodes, SR, Wn_n, Wn_sr, W_sr)
        C2 = _c_update(W_ee, edges, s2_2d)
        edges_flat = _edge_kernel(C2.reshape(-1), AB.reshape(-1), s2, r2)
        edges = edges_flat.reshape(DE, E)
    return nodes, edges, receivers, senders


# speedup vs baseline: 63.3412x; 1.3704x over previous
import functools

import jax
import jax.numpy as jnp
from jax import lax
from jax.experimental import pallas as pl
from jax.experimental.pallas import tpu as pltpu
from jax.experimental.pallas import tpu_sc as plsc

NC = 2
NS = 16
NW = NC * NS
L = 16

DN = 128
DE = 16
N = 10000
E = 320000
K = E - N

MESH = dict(
    mesh=plsc.VectorSubcoreMesh(
        core_axis_name="c", subcore_axis_name="s", num_cores=NC, num_subcores=NS),
    compiler_params=pltpu.CompilerParams(needs_layout_passes=False),
)


def _wid():
    return lax.axis_index("s") * NC + lax.axis_index("c")


_CHM = E // NW


@functools.partial(
    pl.kernel,
    out_type=jax.ShapeDtypeStruct((NW * L,), jnp.int32),
    scratch_types=[
        pltpu.VMEM((_CHM,), jnp.int32),
        pltpu.VMEM((_CHM,), jnp.int32),
        pltpu.VMEM((L,), jnp.int32),
    ],
    **MESH,
)
def _count_kernel(s_hbm, r_hbm, out_hbm, sbuf, rbuf, cbuf):
    w = _wid()
    base = w * _CHM
    pltpu.sync_copy(s_hbm.at[pl.ds(base, _CHM)], sbuf)
    pltpu.sync_copy(r_hbm.at[pl.ds(base, _CHM)], rbuf)

    def body(j, cnt):
        sv = sbuf[pl.ds(j * L, L)]
        rv = rbuf[pl.ds(j * L, L)]
        return cnt + jnp.where(sv != rv, 1, 0).astype(jnp.int32)

    cnt = lax.fori_loop(0, _CHM // L, body, jnp.zeros((L,), jnp.int32))
    cbuf[...] = jnp.broadcast_to(jnp.sum(cnt), (L,))
    pltpu.sync_copy(cbuf, out_hbm.at[pl.ds(w * L, L)])


@functools.partial(
    pl.kernel,
    out_type=(
        jax.ShapeDtypeStruct((E,), jnp.int32),
        jax.ShapeDtypeStruct((E,), jnp.int32),
    ),
    scratch_types=[
        pltpu.VMEM((_CHM,), jnp.int32),
        pltpu.VMEM((_CHM,), jnp.int32),
        pltpu.VMEM((_CHM,), jnp.int32),
        pltpu.VMEM((_CHM,), jnp.int32),
        pltpu.VMEM((NW * L,), jnp.int32),
    ],
    **MESH,
)
def _rewrite_kernel(s_hbm, r_hbm, cnt_hbm, s2_hbm, r2_hbm, sbuf, rbuf, s2buf, r2buf, cntbuf):
    w = _wid()
    base = w * _CHM
    pltpu.sync_copy(s_hbm.at[pl.ds(base, _CHM)], sbuf)
    pltpu.sync_copy(r_hbm.at[pl.ds(base, _CHM)], rbuf)
    pltpu.sync_copy(cnt_hbm, cntbuf)

    off = jnp.zeros((L,), jnp.int32)
    tot = jnp.zeros((L,), jnp.int32)
    for k in range(NW):
        ck = cntbuf[pl.ds(k * L, L)]
        off = off + jnp.where(jnp.broadcast_to(k < w, (L,)), ck, 0)
        tot = tot + ck
    lane = lax.iota(jnp.int32, L)
    pad_force = jnp.broadcast_to((w == 0) & (jnp.sum(tot) // L < K), (L,)) & (lane == 0)

    def body(j, running):
        sv = sbuf[pl.ds(j * L, L)]
        rv = rbuf[pl.ds(j * L, L)]
        m = sv != rv
        mi = jnp.where(m, 1, 0).astype(jnp.int32)
        incl = plsc.cumsum(mi)
        rank = incl + running
        upd = m & (rank <= K)
        upd = upd | (pad_force & jnp.broadcast_to(j == 0, (L,)))
        s2buf[pl.ds(j * L, L)] = jnp.where(upd, sv, N)
        r2buf[pl.ds(j * L, L)] = jnp.where(upd, rv, N)
        return running + jnp.max(incl)

    lax.fori_loop(0, _CHM // L, body, jnp.sum(off) // L)
    pltpu.sync_copy(s2buf, s2_hbm.at[pl.ds(base, _CHM)])
    pltpu.sync_copy(r2buf, r2_hbm.at[pl.ds(base, _CHM)])


_CH1 = 16000


@functools.partial(
    pl.kernel,
    out_type=jax.ShapeDtypeStruct((2 * DE * N,), jnp.float32),
    scratch_types=[
        pltpu.VMEM((N,), jnp.float32),
        pltpu.VMEM((_CH1,), jnp.float32),
        pltpu.VMEM((_CH1,), jnp.float32),
        pltpu.VMEM((_CH1,), jnp.int32),
        pltpu.VMEM((_CH1,), jnp.int32),
        pltpu.SemaphoreType.DMA,
        pltpu.SemaphoreType.DMA,
    ],
    **MESH,
)
def _segsum_kernel(edges_hbm, sridx_hbm, out_hbm, acc, vbuf0, vbuf1, ibuf0, ibuf1,
                   sem0, sem1):
    w = _wid()
    t = w // DE
    f = w % DE
    nch = E // _CH1

    def start(c, vb, ib, sem):
        base = c * _CH1
        pltpu.async_copy(edges_hbm.at[pl.ds(f * E + base, _CH1)], vb, sem)
        pltpu.async_copy(sridx_hbm.at[pl.ds(t * E + base, _CH1)], ib, sem)

    def drain(vb, ib, sem):
        pltpu.make_async_copy(edges_hbm.at[pl.ds(f * E, _CH1)], vb, sem).wait()
        pltpu.make_async_copy(sridx_hbm.at[pl.ds(t * E, _CH1)], ib, sem).wait()

    def process(vb, ib):
        @plsc.parallel_loop(0, _CH1, L, unroll=8)
        def _scat(j):
            iv = ib[pl.ds(j, L)]
            vv = vb[pl.ds(j, L)]
            plsc.addupdate_scatter(acc, [iv], vv)

    start(0, vbuf0, ibuf0, sem0)

    @plsc.parallel_loop(0, N, L, unroll=8)
    def _zero(i):
        acc[pl.ds(i, L)] = jnp.zeros((L,), jnp.float32)

    def pair(p, _):
        c0 = 2 * p
        start(c0 + 1, vbuf1, ibuf1, sem1)
        drain(vbuf0, ibuf0, sem0)
        process(vbuf0, ibuf0)

        @pl.when(c0 + 2 < nch)
        def _():
            start(c0 + 2, vbuf0, ibuf0, sem0)

        drain(vbuf1, ibuf1, sem1)
        process(vbuf1, ibuf1)
        return 0

    lax.fori_loop(0, nch // 2, pair, 0)
    pltpu.sync_copy(acc, out_hbm.at[pl.ds(w * N, N)])


_CH2 = 8000
_HALF = E // 2


@functools.partial(
    pl.kernel,
    out_type=jax.ShapeDtypeStruct((DE * E,), jnp.float32),
    scratch_types=[
        pltpu.VMEM((N + L,), jnp.float32),
        pltpu.VMEM((N + L,), jnp.float32),
        pltpu.VMEM((_CH2,), jnp.float32),
        pltpu.VMEM((_CH2,), jnp.float32),
        pltpu.VMEM((_CH2,), jnp.int32),
        pltpu.VMEM((_CH2,), jnp.int32),
        pltpu.VMEM((_CH2,), jnp.int32),
        pltpu.VMEM((_CH2,), jnp.int32),
        pltpu.VMEM((_CH2,), jnp.float32),
        pltpu.VMEM((_CH2,), jnp.float32),
        pltpu.SemaphoreType.DMA,
        pltpu.SemaphoreType.DMA,
        pltpu.SemaphoreType.DMA,
        pltpu.SemaphoreType.DMA,
    ],
    **MESH,
)
def _edge_kernel(c2_hbm, ab_hbm, s2_hbm, r2_hbm, out_hbm, arow, brow,
                 cbuf0, cbuf1, sbuf0, sbuf1, rbuf0, rbuf1, obuf0, obuf1,
                 sem0, sem1, osem0, osem1):
    w = _wid()
    h = w // DE
    f = w % DE
    nch = _HALF // _CH2

    def start(c, cb, sb, rb, sem):
        base = h * _HALF + c * _CH2
        pltpu.async_copy(c2_hbm.at[pl.ds(f * E + base, _CH2)], cb, sem)
        pltpu.async_copy(s2_hbm.at[pl.ds(base, _CH2)], sb, sem)
        pltpu.async_copy(r2_hbm.at[pl.ds(base, _CH2)], rb, sem)

    def drain(cb, sb, rb, sem):
        pltpu.make_async_copy(c2_hbm.at[pl.ds(0, _CH2)], cb, sem).wait()
        pltpu.make_async_copy(s2_hbm.at[pl.ds(0, _CH2)], sb, sem).wait()
        pltpu.make_async_copy(r2_hbm.at[pl.ds(0, _CH2)], rb, sem).wait()

    def process(c, cb, sb, rb, ob, osem):
        @plsc.parallel_loop(0, _CH2, L, unroll=8)
        def _gat(j):
            sv = sb[pl.ds(j, L)]
            rv = rb[pl.ds(j, L)]
            cv = cb[pl.ds(j, L)]
            av = plsc.load_gather(arow, [sv])
            bv = plsc.load_gather(brow, [rv])
            ob[pl.ds(j, L)] = cv + av + bv

        base = h * _HALF + c * _CH2
        pltpu.async_copy(ob, out_hbm.at[pl.ds(f * E + base, _CH2)], osem)

    def owait(ob, osem):
        pltpu.make_async_copy(ob, out_hbm.at[pl.ds(0, _CH2)], osem).wait()

    start(0, cbuf0, sbuf0, rbuf0, sem0)
    pltpu.sync_copy(ab_hbm.at[pl.ds(f * N, N)], arow.at[pl.ds(0, N)])
    pltpu.sync_copy(ab_hbm.at[pl.ds((DE + f) * N, N)], brow.at[pl.ds(0, N)])
    arow[pl.ds(N, L)] = jnp.zeros((L,), jnp.float32)
    brow[pl.ds(N, L)] = jnp.zeros((L,), jnp.float32)

    def pair(p, _):
        c0 = 2 * p
        start(c0 + 1, cbuf1, sbuf1, rbuf1, sem1)
        drain(cbuf0, sbuf0, rbuf0, sem0)

        @pl.when(c0 >= 2)
        def _():
            owait(obuf0, osem0)

        process(c0, cbuf0, sbuf0, rbuf0, obuf0, osem0)

        @pl.when(c0 + 2 < nch)
        def _():
            start(c0 + 2, cbuf0, sbuf0, rbuf0, sem0)

        drain(cbuf1, sbuf1, rbuf1, sem1)

        @pl.when(c0 >= 2)
        def _():
            owait(obuf1, osem1)

        process(c0 + 1, cbuf1, sbuf1, rbuf1, obuf1, osem1)
        return 0

    lax.fori_loop(0, nch // 2, pair, 0)
    owait(obuf0, osem0)
    owait(obuf1, osem1)


def _node_body(nodes_ref, sr_ref, wnn_ref, wnsr_ref, wsr_ref, nodes_out, ab_out):
    dot = functools.partial(
        jnp.dot, precision=lax.Precision.HIGHEST, preferred_element_type=jnp.float32
    )
    nn = dot(wnn_ref[...], nodes_ref[...]) + dot(wnsr_ref[...], sr_ref[...])
    nodes_out[...] = nn
    ab_out[...] = dot(wsr_ref[...], nn)


def _node_update(nodes, SR, Wn_n, Wn_sr, W_sr):
    return pl.pallas_call(
        _node_body,
        out_shape=(
            jax.ShapeDtypeStruct((DN, N), jnp.float32),
            jax.ShapeDtypeStruct((2 * DE, N), jnp.float32),
        ),
    )(nodes, SR, Wn_n, Wn_sr, W_sr)


_BC = 16000


def _c_body(wee_ref, e_ref, s2_ref, out_ref):
    c = jnp.dot(
        wee_ref[...], e_ref[...],
        precision=lax.Precision.HIGHEST, preferred_element_type=jnp.float32,
    )
    out_ref[...] = jnp.where(s2_ref[...] != N, c, e_ref[...])


def _c_update(W_ee, edges, s2_2d):
    return pl.pallas_call(
        _c_body,
        grid=(E // _BC,),
        in_specs=[
            pl.BlockSpec((DE, DE), lambda i: (0, 0)),
            pl.BlockSpec((DE, _BC), lambda i: (0, i)),
            pl.BlockSpec((1, _BC), lambda i: (0, i)),
        ],
        out_specs=pl.BlockSpec((DE, _BC), lambda i: (0, i)),
        out_shape=jax.ShapeDtypeStruct((DE, E), jnp.float32),
    )(W_ee, edges, s2_2d)


def kernel(nodes, edges, receivers, senders, W_node, W_edge):
    Wn_n = W_node[:, :DN]
    Wn_sr = W_node[:, DN:DN + 2 * DE]
    W_ee = W_edge[:, :DE]
    W_sr = jnp.concatenate([W_edge[:, DE:DE + DN], W_edge[:, DE + DN:]], axis=0)
    sr_idx = jnp.concatenate([senders, receivers])

    counts = _count_kernel(senders, receivers)
    s2, r2 = _rewrite_kernel(senders, receivers, counts)
    s2_2d = s2.reshape(1, E)

    edges_flat = edges.reshape(-1)
    for _ in range(2):
        SR = _segsum_kernel(edges_flat, sr_idx).reshape(2 * DE, N)
        nodes, AB = _node_update(nodes, SR, Wn_n, Wn_sr, W_sr)
        C2 = _c_update(W_ee, edges, s2_2d)
        edges_flat = _edge_kernel(C2.reshape(-1), AB.reshape(-1), s2, r2)
        edges = edges_flat.reshape(DE, E)
    return nodes, edges, receivers, senders


# speedup vs baseline: 66.0689x; 1.0431x over previous
import functools

import jax
import jax.numpy as jnp
from jax import lax
from jax.experimental import pallas as pl
from jax.experimental.pallas import tpu as pltpu
from jax.experimental.pallas import tpu_sc as plsc

NC = 2
NS = 16
NW = NC * NS
L = 16

DN = 128
DE = 16
N = 10000
E = 320000
K = E - N
NP = 10112

MESH = dict(
    mesh=plsc.VectorSubcoreMesh(
        core_axis_name="c", subcore_axis_name="s", num_cores=NC, num_subcores=NS),
    compiler_params=pltpu.CompilerParams(needs_layout_passes=False),
)


def _wid():
    return lax.axis_index("s") * NC + lax.axis_index("c")


_CHM = E // NW


@functools.partial(
    pl.kernel,
    out_type=jax.ShapeDtypeStruct((NW * L,), jnp.int32),
    scratch_types=[
        pltpu.VMEM((_CHM,), jnp.int32),
        pltpu.VMEM((_CHM,), jnp.int32),
        pltpu.VMEM((L,), jnp.int32),
    ],
    **MESH,
)
def _count_kernel(s_hbm, r_hbm, out_hbm, sbuf, rbuf, cbuf):
    w = _wid()
    base = w * _CHM
    pltpu.sync_copy(s_hbm.at[pl.ds(base, _CHM)], sbuf)
    pltpu.sync_copy(r_hbm.at[pl.ds(base, _CHM)], rbuf)

    def body(j, cnt):
        sv = sbuf[pl.ds(j * L, L)]
        rv = rbuf[pl.ds(j * L, L)]
        return cnt + jnp.where(sv != rv, 1, 0).astype(jnp.int32)

    cnt = lax.fori_loop(0, _CHM // L, body, jnp.zeros((L,), jnp.int32))
    cbuf[...] = jnp.broadcast_to(jnp.sum(cnt), (L,))
    pltpu.sync_copy(cbuf, out_hbm.at[pl.ds(w * L, L)])


@functools.partial(
    pl.kernel,
    out_type=(
        jax.ShapeDtypeStruct((E,), jnp.int32),
        jax.ShapeDtypeStruct((E,), jnp.int32),
    ),
    scratch_types=[
        pltpu.VMEM((_CHM,), jnp.int32),
        pltpu.VMEM((_CHM,), jnp.int32),
        pltpu.VMEM((_CHM,), jnp.int32),
        pltpu.VMEM((_CHM,), jnp.int32),
        pltpu.VMEM((NW * L,), jnp.int32),
    ],
    **MESH,
)
def _rewrite_kernel(s_hbm, r_hbm, cnt_hbm, s2_hbm, r2_hbm, sbuf, rbuf, s2buf, r2buf, cntbuf):
    w = _wid()
    base = w * _CHM
    pltpu.sync_copy(s_hbm.at[pl.ds(base, _CHM)], sbuf)
    pltpu.sync_copy(r_hbm.at[pl.ds(base, _CHM)], rbuf)
    pltpu.sync_copy(cnt_hbm, cntbuf)

    off = jnp.zeros((L,), jnp.int32)
    tot = jnp.zeros((L,), jnp.int32)
    for k in range(NW):
        ck = cntbuf[pl.ds(k * L, L)]
        off = off + jnp.where(jnp.broadcast_to(k < w, (L,)), ck, 0)
        tot = tot + ck
    lane = lax.iota(jnp.int32, L)
    pad_force = jnp.broadcast_to((w == 0) & (jnp.sum(tot) // L < K), (L,)) & (lane == 0)

    def body(j, running):
        sv = sbuf[pl.ds(j * L, L)]
        rv = rbuf[pl.ds(j * L, L)]
        m = sv != rv
        mi = jnp.where(m, 1, 0).astype(jnp.int32)
        incl = plsc.cumsum(mi)
        rank = incl + running
        upd = m & (rank <= K)
        upd = upd | (pad_force & jnp.broadcast_to(j == 0, (L,)))
        s2buf[pl.ds(j * L, L)] = jnp.where(upd, sv, N)
        r2buf[pl.ds(j * L, L)] = jnp.where(upd, rv, N)
        return running + jnp.max(incl)

    lax.fori_loop(0, _CHM // L, body, jnp.sum(off) // L)
    pltpu.sync_copy(s2buf, s2_hbm.at[pl.ds(base, _CHM)])
    pltpu.sync_copy(r2buf, r2_hbm.at[pl.ds(base, _CHM)])


_CH1 = 16000


@functools.partial(
    pl.kernel,
    out_type=jax.ShapeDtypeStruct((2 * DE * NP,), jnp.float32),
    scratch_types=[
        pltpu.VMEM((N,), jnp.float32),
        pltpu.VMEM((_CH1,), jnp.float32),
        pltpu.VMEM((_CH1,), jnp.float32),
        pltpu.VMEM((_CH1,), jnp.int32),
        pltpu.VMEM((_CH1,), jnp.int32),
        pltpu.SemaphoreType.DMA,
        pltpu.SemaphoreType.DMA,
    ],
    **MESH,
)
def _segsum_kernel(edges_hbm, sridx_hbm, out_hbm, acc, vbuf0, vbuf1, ibuf0, ibuf1,
                   sem0, sem1):
    w = _wid()
    t = w // DE
    f = w % DE
    nch = E // _CH1

    def start(c, vb, ib, sem):
        base = c * _CH1
        pltpu.async_copy(edges_hbm.at[pl.ds(f * E + base, _CH1)], vb, sem)
        pltpu.async_copy(sridx_hbm.at[pl.ds(t * E + base, _CH1)], ib, sem)

    def drain(vb, ib, sem):
        pltpu.make_async_copy(edges_hbm.at[pl.ds(f * E, _CH1)], vb, sem).wait()
        pltpu.make_async_copy(sridx_hbm.at[pl.ds(t * E, _CH1)], ib, sem).wait()

    def process(vb, ib):
        @plsc.parallel_loop(0, _CH1, L, unroll=8)
        def _scat(j):
            iv = ib[pl.ds(j, L)]
            vv = vb[pl.ds(j, L)]
            plsc.addupdate_scatter(acc, [iv], vv)

    start(0, vbuf0, ibuf0, sem0)

    @plsc.parallel_loop(0, N, L, unroll=8)
    def _zero(i):
        acc[pl.ds(i, L)] = jnp.zeros((L,), jnp.float32)

    def pair(p, _):
        c0 = 2 * p
        start(c0 + 1, vbuf1, ibuf1, sem1)
        drain(vbuf0, ibuf0, sem0)
        process(vbuf0, ibuf0)

        @pl.when(c0 + 2 < nch)
        def _():
            start(c0 + 2, vbuf0, ibuf0, sem0)

        drain(vbuf1, ibuf1, sem1)
        process(vbuf1, ibuf1)
        return 0

    lax.fori_loop(0, nch // 2, pair, 0)
    pltpu.sync_copy(acc, out_hbm.at[pl.ds(w * NP, N)])


_CH2 = 8000
_HALF = E // 2


@functools.partial(
    pl.kernel,
    out_type=jax.ShapeDtypeStruct((DE * E,), jnp.float32),
    scratch_types=[
        pltpu.VMEM((N + L,), jnp.float32),
        pltpu.VMEM((N + L,), jnp.float32),
        pltpu.VMEM((_CH2,), jnp.float32),
        pltpu.VMEM((_CH2,), jnp.float32),
        pltpu.VMEM((_CH2,), jnp.int32),
        pltpu.VMEM((_CH2,), jnp.int32),
        pltpu.VMEM((_CH2,), jnp.int32),
        pltpu.VMEM((_CH2,), jnp.int32),
        pltpu.VMEM((_CH2,), jnp.float32),
        pltpu.VMEM((_CH2,), jnp.float32),
        pltpu.SemaphoreType.DMA,
        pltpu.SemaphoreType.DMA,
        pltpu.SemaphoreType.DMA,
        pltpu.SemaphoreType.DMA,
    ],
    **MESH,
)
def _edge_kernel(c2_hbm, ab_hbm, s2_hbm, r2_hbm, out_hbm, arow, brow,
                 cbuf0, cbuf1, sbuf0, sbuf1, rbuf0, rbuf1, obuf0, obuf1,
                 sem0, sem1, osem0, osem1):
    w = _wid()
    h = w // DE
    f = w % DE
    nch = _HALF // _CH2

    def start(c, cb, sb, rb, sem):
        base = h * _HALF + c * _CH2
        pltpu.async_copy(c2_hbm.at[pl.ds(f * E + base, _CH2)], cb, sem)
        pltpu.async_copy(s2_hbm.at[pl.ds(base, _CH2)], sb, sem)
        pltpu.async_copy(r2_hbm.at[pl.ds(base, _CH2)], rb, sem)

    def drain(cb, sb, rb, sem):
        pltpu.make_async_copy(c2_hbm.at[pl.ds(0, _CH2)], cb, sem).wait()
        pltpu.make_async_copy(s2_hbm.at[pl.ds(0, _CH2)], sb, sem).wait()
        pltpu.make_async_copy(r2_hbm.at[pl.ds(0, _CH2)], rb, sem).wait()

    def process(c, cb, sb, rb, ob, osem):
        @plsc.parallel_loop(0, _CH2, L, unroll=8)
        def _gat(j):
            sv = sb[pl.ds(j, L)]
            rv = rb[pl.ds(j, L)]
            cv = cb[pl.ds(j, L)]
            av = plsc.load_gather(arow, [sv])
            bv = plsc.load_gather(brow, [rv])
            ob[pl.ds(j, L)] = cv + av + bv

        base = h * _HALF + c * _CH2
        pltpu.async_copy(ob, out_hbm.at[pl.ds(f * E + base, _CH2)], osem)

    def owait(ob, osem):
        pltpu.make_async_copy(ob, out_hbm.at[pl.ds(0, _CH2)], osem).wait()

    start(0, cbuf0, sbuf0, rbuf0, sem0)
    pltpu.sync_copy(ab_hbm.at[pl.ds(f * NP, N)], arow.at[pl.ds(0, N)])
    pltpu.sync_copy(ab_hbm.at[pl.ds((DE + f) * NP, N)], brow.at[pl.ds(0, N)])
    arow[pl.ds(N, L)] = jnp.zeros((L,), jnp.float32)
    brow[pl.ds(N, L)] = jnp.zeros((L,), jnp.float32)

    def pair(p, _):
        c0 = 2 * p
        start(c0 + 1, cbuf1, sbuf1, rbuf1, sem1)
        drain(cbuf0, sbuf0, rbuf0, sem0)

        @pl.when(c0 >= 2)
        def _():
            owait(obuf0, osem0)

        process(c0, cbuf0, sbuf0, rbuf0, obuf0, osem0)

        @pl.when(c0 + 2 < nch)
        def _():
            start(c0 + 2, cbuf0, sbuf0, rbuf0, sem0)

        drain(cbuf1, sbuf1, rbuf1, sem1)

        @pl.when(c0 >= 2)
        def _():
            owait(obuf1, osem1)

        process(c0 + 1, cbuf1, sbuf1, rbuf1, obuf1, osem1)
        return 0

    lax.fori_loop(0, nch // 2, pair, 0)
    owait(obuf0, osem0)
    owait(obuf1, osem1)


_BC = 16000
_NB = E // _BC


def _m13_body(nodes_ref, wnn_ref, wnsr_ref, wsr_ref, wee_ref,
              sr_hbm, e_hbm, s2_hbm,
              nodes_out, ab_hbm, c2_hbm,
              sr_v, ab_v, e0, e1, c0, c1, s0, s1,
              sem_sr, sem_ab, semi0, semi1, semo0, semo1):
    dot = functools.partial(
        jnp.dot, precision=lax.Precision.HIGHEST, preferred_element_type=jnp.float32
    )

    def start_in(c, eb, sb, sem):
        base = c * _BC
        for ff in range(DE):
            pltpu.async_copy(e_hbm.at[pl.ds(ff * E + base, _BC)], eb.at[ff], sem)
        pltpu.async_copy(s2_hbm.at[pl.ds(base, _BC)], sb, sem)

    def drain_in(eb, sb, sem):
        for ff in range(DE):
            pltpu.make_async_copy(e_hbm.at[pl.ds(0, _BC)], eb.at[ff], sem).wait()
        pltpu.make_async_copy(s2_hbm.at[pl.ds(0, _BC)], sb, sem).wait()

    def start_out(c, cb, sem):
        base = c * _BC
        for ff in range(DE):
            pltpu.async_copy(cb.at[ff], c2_hbm.at[pl.ds(ff * E + base, _BC)], sem)

    def drain_out(cb, sem):
        for ff in range(DE):
            pltpu.make_async_copy(cb.at[ff], c2_hbm.at[pl.ds(0, _BC)], sem).wait()

    start_in(0, e0, s0, semi0)
    start_in(1, e1, s1, semi1)

    for rr in range(2 * DE):
        pltpu.async_copy(sr_hbm.at[pl.ds(rr * NP, NP)], sr_v.at[rr], sem_sr)
    for rr in range(2 * DE):
        pltpu.make_async_copy(sr_hbm.at[pl.ds(0, NP)], sr_v.at[rr], sem_sr).wait()
    nn = (dot(wnn_ref[...], nodes_ref[...])
          + dot(wnsr_ref[...], sr_v[...])[:, :N])
    nodes_out[...] = nn
    ab_v[:, pl.ds(0, N)] = dot(wsr_ref[...], nn)
    for rr in range(2 * DE):
        pltpu.async_copy(ab_v.at[rr], ab_hbm.at[pl.ds(rr * NP, NP)], sem_ab)

    for c in range(_NB):
        eb, sb, cb, semi, semo = ((e0, s0, c0, semi0, semo0) if c % 2 == 0
                                  else (e1, s1, c1, semi1, semo1))
        drain_in(eb, sb, semi)
        if c >= 2:
            drain_out(cb, semo)
        e = eb[...]
        cm = dot(wee_ref[...], e)
        u = (sb[...] != N).reshape(1, _BC)
        cb[...] = jnp.where(u, cm, e)
        start_out(c, cb, semo)
        if c + 2 < _NB:
            start_in(c + 2, eb, sb, semi)

    drain_out(c0 if _NB % 2 == 0 else c1, semo0 if _NB % 2 == 0 else semo1)
    drain_out(c1 if _NB % 2 == 0 else c0, semo1 if _NB % 2 == 0 else semo0)
    for rr in range(2 * DE):
        pltpu.make_async_copy(ab_v.at[rr], ab_hbm.at[pl.ds(0, NP)], sem_ab).wait()


def _m13(nodes, SR_flat, edges_flat, s2, Wn_n, Wn_sr, W_sr, W_ee):
    hbm = pl.BlockSpec(memory_space=pltpu.MemorySpace.HBM)
    vmem = pl.BlockSpec(memory_space=pltpu.MemorySpace.VMEM)
    return pl.pallas_call(
        _m13_body,
        in_specs=[vmem, vmem, vmem, vmem, vmem, hbm, hbm, hbm],
        out_specs=(vmem, hbm, hbm),
        out_shape=(
            jax.ShapeDtypeStruct((DN, N), jnp.float32),
            jax.ShapeDtypeStruct((2 * DE * NP,), jnp.float32),
            jax.ShapeDtypeStruct((DE * E,), jnp.float32),
        ),
        scratch_shapes=[
            pltpu.VMEM((2 * DE, NP), jnp.float32),
            pltpu.VMEM((2 * DE, NP), jnp.float32),
            pltpu.VMEM((DE, _BC), jnp.float32),
            pltpu.VMEM((DE, _BC), jnp.float32),
            pltpu.VMEM((DE, _BC), jnp.float32),
            pltpu.VMEM((DE, _BC), jnp.float32),
            pltpu.VMEM((_BC,), jnp.int32),
            pltpu.VMEM((_BC,), jnp.int32),
            pltpu.SemaphoreType.DMA,
            pltpu.SemaphoreType.DMA,
            pltpu.SemaphoreType.DMA,
            pltpu.SemaphoreType.DMA,
            pltpu.SemaphoreType.DMA,
            pltpu.SemaphoreType.DMA,
        ],
    )(nodes, Wn_n, Wn_sr, W_sr, W_ee, SR_flat, edges_flat, s2)


def kernel(nodes, edges, receivers, senders, W_node, W_edge):
    Wn_n = W_node[:, :DN]
    Wn_sr = W_node[:, DN:DN + 2 * DE]
    W_ee = W_edge[:, :DE]
    W_sr = jnp.concatenate([W_edge[:, DE:DE + DN], W_edge[:, DE + DN:]], axis=0)
    sr_idx = jnp.concatenate([senders, receivers])

    counts = _count_kernel(senders, receivers)
    s2, r2 = _rewrite_kernel(senders, receivers, counts)

    edges_flat = edges.reshape(-1)
    for _ in range(2):
        SR_flat = _segsum_kernel(edges_flat, sr_idx)
        nodes, AB_flat, C2_flat = _m13(
            nodes, SR_flat, edges_flat, s2, Wn_n, Wn_sr, W_sr, W_ee)
        edges_flat = _edge_kernel(C2_flat, AB_flat, s2, r2)
    return nodes, edges_flat.reshape(DE, E), receivers, senders


# speedup vs baseline: 66.4230x; 1.0054x over previous
import functools

import jax
import jax.numpy as jnp
from jax import lax
from jax.experimental import pallas as pl
from jax.experimental.pallas import tpu as pltpu
from jax.experimental.pallas import tpu_sc as plsc

NC = 2
NS = 16
NW = NC * NS
L = 16

DN = 128
DE = 16
N = 10000
E = 320000
K = E - N
NP = 10112

MESH = dict(
    mesh=plsc.VectorSubcoreMesh(
        core_axis_name="c", subcore_axis_name="s", num_cores=NC, num_subcores=NS),
    compiler_params=pltpu.CompilerParams(needs_layout_passes=False),
)


def _wid():
    return lax.axis_index("s") * NC + lax.axis_index("c")


_CHM = E // NW


@functools.partial(
    pl.kernel,
    out_type=jax.ShapeDtypeStruct((NW * L,), jnp.int32),
    scratch_types=[
        pltpu.VMEM((_CHM,), jnp.int32),
        pltpu.VMEM((_CHM,), jnp.int32),
        pltpu.VMEM((L,), jnp.int32),
    ],
    **MESH,
)
def _count_kernel(s_hbm, r_hbm, out_hbm, sbuf, rbuf, cbuf):
    w = _wid()
    base = w * _CHM
    pltpu.sync_copy(s_hbm.at[pl.ds(base, _CHM)], sbuf)
    pltpu.sync_copy(r_hbm.at[pl.ds(base, _CHM)], rbuf)

    def body(j, cnt):
        sv = sbuf[pl.ds(j * L, L)]
        rv = rbuf[pl.ds(j * L, L)]
        return cnt + jnp.where(sv != rv, 1, 0).astype(jnp.int32)

    cnt = lax.fori_loop(0, _CHM // L, body, jnp.zeros((L,), jnp.int32))
    cbuf[...] = jnp.broadcast_to(jnp.sum(cnt), (L,))
    pltpu.sync_copy(cbuf, out_hbm.at[pl.ds(w * L, L)])


@functools.partial(
    pl.kernel,
    out_type=(
        jax.ShapeDtypeStruct((E,), jnp.int32),
        jax.ShapeDtypeStruct((E,), jnp.int32),
    ),
    scratch_types=[
        pltpu.VMEM((_CHM,), jnp.int32),
        pltpu.VMEM((_CHM,), jnp.int32),
        pltpu.VMEM((_CHM,), jnp.int32),
        pltpu.VMEM((_CHM,), jnp.int32),
        pltpu.VMEM((NW * L,), jnp.int32),
    ],
    **MESH,
)
def _rewrite_kernel(s_hbm, r_hbm, cnt_hbm, s2_hbm, r2_hbm, sbuf, rbuf, s2buf, r2buf, cntbuf):
    w = _wid()
    base = w * _CHM
    pltpu.sync_copy(s_hbm.at[pl.ds(base, _CHM)], sbuf)
    pltpu.sync_copy(r_hbm.at[pl.ds(base, _CHM)], rbuf)
    pltpu.sync_copy(cnt_hbm, cntbuf)

    off = jnp.zeros((L,), jnp.int32)
    tot = jnp.zeros((L,), jnp.int32)
    for k in range(NW):
        ck = cntbuf[pl.ds(k * L, L)]
        off = off + jnp.where(jnp.broadcast_to(k < w, (L,)), ck, 0)
        tot = tot + ck
    lane = lax.iota(jnp.int32, L)
    pad_force = jnp.broadcast_to((w == 0) & (jnp.sum(tot) // L < K), (L,)) & (lane == 0)

    def body(j, running):
        sv = sbuf[pl.ds(j * L, L)]
        rv = rbuf[pl.ds(j * L, L)]
        m = sv != rv
        mi = jnp.where(m, 1, 0).astype(jnp.int32)
        incl = plsc.cumsum(mi)
        rank = incl + running
        upd = m & (rank <= K)
        upd = upd | (pad_force & jnp.broadcast_to(j == 0, (L,)))
        s2buf[pl.ds(j * L, L)] = jnp.where(upd, sv, N)
        r2buf[pl.ds(j * L, L)] = jnp.where(upd, rv, N)
        return running + jnp.max(incl)

    lax.fori_loop(0, _CHM // L, body, jnp.sum(off) // L)
    pltpu.sync_copy(s2buf, s2_hbm.at[pl.ds(base, _CHM)])
    pltpu.sync_copy(r2buf, r2_hbm.at[pl.ds(base, _CHM)])


_CH1 = 16000


@functools.partial(
    pl.kernel,
    out_type=jax.ShapeDtypeStruct((2 * DE * NP,), jnp.float32),
    scratch_types=[
        pltpu.VMEM((N,), jnp.float32),
        pltpu.VMEM((_CH1,), jnp.float32),
        pltpu.VMEM((_CH1,), jnp.float32),
        pltpu.VMEM((_CH1,), jnp.int32),
        pltpu.VMEM((_CH1,), jnp.int32),
        pltpu.SemaphoreType.DMA,
        pltpu.SemaphoreType.DMA,
    ],
    **MESH,
)
def _segsum_kernel(edges_hbm, sridx_hbm, out_hbm, acc, vbuf0, vbuf1, ibuf0, ibuf1,
                   sem0, sem1):
    w = _wid()
    t = w // DE
    f = w % DE
    nch = E // _CH1

    def start(c, vb, ib, sem):
        base = c * _CH1
        pltpu.async_copy(edges_hbm.at[pl.ds(f * E + base, _CH1)], vb, sem)
        pltpu.async_copy(sridx_hbm.at[pl.ds(t * E + base, _CH1)], ib, sem)

    def drain(vb, ib, sem):
        pltpu.make_async_copy(edges_hbm.at[pl.ds(f * E, _CH1)], vb, sem).wait()
        pltpu.make_async_copy(sridx_hbm.at[pl.ds(t * E, _CH1)], ib, sem).wait()

    def process(vb, ib):
        @plsc.parallel_loop(0, _CH1, L, unroll=8)
        def _scat(j):
            iv = ib[pl.ds(j, L)]
            vv = vb[pl.ds(j, L)]
            plsc.addupdate_scatter(acc, [iv], vv)

    start(0, vbuf0, ibuf0, sem0)

    @plsc.parallel_loop(0, N, L, unroll=8)
    def _zero(i):
        acc[pl.ds(i, L)] = jnp.zeros((L,), jnp.float32)

    def pair(p, _):
        c0 = 2 * p
        start(c0 + 1, vbuf1, ibuf1, sem1)
        drain(vbuf0, ibuf0, sem0)
        process(vbuf0, ibuf0)

        @pl.when(c0 + 2 < nch)
        def _():
            start(c0 + 2, vbuf0, ibuf0, sem0)

        drain(vbuf1, ibuf1, sem1)
        process(vbuf1, ibuf1)
        return 0

    lax.fori_loop(0, nch // 2, pair, 0)
    pltpu.sync_copy(acc, out_hbm.at[pl.ds(w * NP, N)])


_CH2 = 8000
_HALF = E // 2


@functools.partial(
    pl.kernel,
    out_type=jax.ShapeDtypeStruct((DE * E,), jnp.float32),
    scratch_types=[
        pltpu.VMEM((N + L,), jnp.float32),
        pltpu.VMEM((N + L,), jnp.float32),
        pltpu.VMEM((_CH2,), jnp.float32),
        pltpu.VMEM((_CH2,), jnp.float32),
        pltpu.VMEM((_CH2,), jnp.int32),
        pltpu.VMEM((_CH2,), jnp.int32),
        pltpu.VMEM((_CH2,), jnp.int32),
        pltpu.VMEM((_CH2,), jnp.int32),
        pltpu.VMEM((_CH2,), jnp.float32),
        pltpu.VMEM((_CH2,), jnp.float32),
        pltpu.SemaphoreType.DMA,
        pltpu.SemaphoreType.DMA,
        pltpu.SemaphoreType.DMA,
        pltpu.SemaphoreType.DMA,
    ],
    **MESH,
)
def _edge_kernel(c2_hbm, ab_hbm, s2_hbm, r2_hbm, out_hbm, arow, brow,
                 cbuf0, cbuf1, sbuf0, sbuf1, rbuf0, rbuf1, obuf0, obuf1,
                 sem0, sem1, osem0, osem1):
    w = _wid()
    h = w // DE
    f = w % DE
    nch = _HALF // _CH2

    def start(c, cb, sb, rb, sem):
        base = h * _HALF + c * _CH2
        pltpu.async_copy(c2_hbm.at[pl.ds(f * E + base, _CH2)], cb, sem)
        pltpu.async_copy(s2_hbm.at[pl.ds(base, _CH2)], sb, sem)
        pltpu.async_copy(r2_hbm.at[pl.ds(base, _CH2)], rb, sem)

    def drain(cb, sb, rb, sem):
        pltpu.make_async_copy(c2_hbm.at[pl.ds(0, _CH2)], cb, sem).wait()
        pltpu.make_async_copy(s2_hbm.at[pl.ds(0, _CH2)], sb, sem).wait()
        pltpu.make_async_copy(r2_hbm.at[pl.ds(0, _CH2)], rb, sem).wait()

    def process(c, cb, sb, rb, ob, osem):
        @plsc.parallel_loop(0, _CH2, L, unroll=8)
        def _gat(j):
            sv = sb[pl.ds(j, L)]
            rv = rb[pl.ds(j, L)]
            cv = cb[pl.ds(j, L)]
            av = plsc.load_gather(arow, [sv])
            bv = plsc.load_gather(brow, [rv])
            ob[pl.ds(j, L)] = cv + av + bv

        base = h * _HALF + c * _CH2
        pltpu.async_copy(ob, out_hbm.at[pl.ds(f * E + base, _CH2)], osem)

    def owait(ob, osem):
        pltpu.make_async_copy(ob, out_hbm.at[pl.ds(0, _CH2)], osem).wait()

    start(0, cbuf0, sbuf0, rbuf0, sem0)
    pltpu.sync_copy(ab_hbm.at[pl.ds(f * NP, N)], arow.at[pl.ds(0, N)])
    pltpu.sync_copy(ab_hbm.at[pl.ds((DE + f) * NP, N)], brow.at[pl.ds(0, N)])
    arow[pl.ds(N, L)] = jnp.zeros((L,), jnp.float32)
    brow[pl.ds(N, L)] = jnp.zeros((L,), jnp.float32)

    def pair(p, _):
        c0 = 2 * p
        start(c0 + 1, cbuf1, sbuf1, rbuf1, sem1)
        drain(cbuf0, sbuf0, rbuf0, sem0)

        @pl.when(c0 >= 2)
        def _():
            owait(obuf0, osem0)

        process(c0, cbuf0, sbuf0, rbuf0, obuf0, osem0)

        @pl.when(c0 + 2 < nch)
        def _():
            start(c0 + 2, cbuf0, sbuf0, rbuf0, sem0)

        drain(cbuf1, sbuf1, rbuf1, sem1)

        @pl.when(c0 >= 2)
        def _():
            owait(obuf1, osem1)

        process(c0 + 1, cbuf1, sbuf1, rbuf1, obuf1, osem1)
        return 0

    lax.fori_loop(0, nch // 2, pair, 0)
    owait(obuf0, osem0)
    owait(obuf1, osem1)


_BC = 32000
_NB = E // _BC


def _m13_body(nodes_ref, wnn_ref, wnsr_ref, wsr_ref, wee_ref,
              sr_hbm, e_hbm, s2_hbm,
              nodes_out, ab_hbm, c2_hbm,
              sr_v, ab_v, e0, e1, c0, c1, s0, s1,
              sem_sr, sem_ab, semi0, semi1, semo0, semo1):
    dot = functools.partial(
        jnp.dot, precision=lax.Precision.HIGHEST, preferred_element_type=jnp.float32
    )

    def start_in(c, eb, sb, sem):
        base = c * _BC
        for ff in range(DE):
            pltpu.async_copy(e_hbm.at[pl.ds(ff * E + base, _BC)], eb.at[ff], sem)
        pltpu.async_copy(s2_hbm.at[pl.ds(base, _BC)], sb, sem)

    def drain_in(eb, sb, sem):
        for ff in range(DE):
            pltpu.make_async_copy(e_hbm.at[pl.ds(0, _BC)], eb.at[ff], sem).wait()
        pltpu.make_async_copy(s2_hbm.at[pl.ds(0, _BC)], sb, sem).wait()

    def start_out(c, cb, sem):
        base = c * _BC
        for ff in range(DE):
            pltpu.async_copy(cb.at[ff], c2_hbm.at[pl.ds(ff * E + base, _BC)], sem)

    def drain_out(cb, sem):
        for ff in range(DE):
            pltpu.make_async_copy(cb.at[ff], c2_hbm.at[pl.ds(0, _BC)], sem).wait()

    start_in(0, e0, s0, semi0)
    start_in(1, e1, s1, semi1)

    for rr in range(2 * DE):
        pltpu.async_copy(sr_hbm.at[pl.ds(rr * NP, NP)], sr_v.at[rr], sem_sr)
    for rr in range(2 * DE):
        pltpu.make_async_copy(sr_hbm.at[pl.ds(0, NP)], sr_v.at[rr], sem_sr).wait()
    nn = (dot(wnn_ref[...], nodes_ref[...])
          + dot(wnsr_ref[...], sr_v[...])[:, :N])
    nodes_out[...] = nn
    ab_v[:, pl.ds(0, N)] = dot(wsr_ref[...], nn)
    for rr in range(2 * DE):
        pltpu.async_copy(ab_v.at[rr], ab_hbm.at[pl.ds(rr * NP, NP)], sem_ab)

    for c in range(_NB):
        eb, sb, cb, semi, semo = ((e0, s0, c0, semi0, semo0) if c % 2 == 0
                                  else (e1, s1, c1, semi1, semo1))
        drain_in(eb, sb, semi)
        if c >= 2:
            drain_out(cb, semo)
        e = eb[...]
        cm = dot(wee_ref[...], e)
        u = (sb[...] != N).reshape(1, _BC)
        cb[...] = jnp.where(u, cm, e)
        start_out(c, cb, semo)
        if c + 2 < _NB:
            start_in(c + 2, eb, sb, semi)

    drain_out(c0 if _NB % 2 == 0 else c1, semo0 if _NB % 2 == 0 else semo1)
    drain_out(c1 if _NB % 2 == 0 else c0, semo1 if _NB % 2 == 0 else semo0)
    for rr in range(2 * DE):
        pltpu.make_async_copy(ab_v.at[rr], ab_hbm.at[pl.ds(0, NP)], sem_ab).wait()


def _m13(nodes, SR_flat, edges_flat, s2, Wn_n, Wn_sr, W_sr, W_ee):
    hbm = pl.BlockSpec(memory_space=pltpu.MemorySpace.HBM)
    vmem = pl.BlockSpec(memory_space=pltpu.MemorySpace.VMEM)
    return pl.pallas_call(
        _m13_body,
        in_specs=[vmem, vmem, vmem, vmem, vmem, hbm, hbm, hbm],
        out_specs=(vmem, hbm, hbm),
        out_shape=(
            jax.ShapeDtypeStruct((DN, N), jnp.float32),
            jax.ShapeDtypeStruct((2 * DE * NP,), jnp.float32),
            jax.ShapeDtypeStruct((DE * E,), jnp.float32),
        ),
        scratch_shapes=[
            pltpu.VMEM((2 * DE, NP), jnp.float32),
            pltpu.VMEM((2 * DE, NP), jnp.float32),
            pltpu.VMEM((DE, _BC), jnp.float32),
            pltpu.VMEM((DE, _BC), jnp.float32),
            pltpu.VMEM((DE, _BC), jnp.float32),
            pltpu.VMEM((DE, _BC), jnp.float32),
            pltpu.VMEM((_BC,), jnp.int32),
            pltpu.VMEM((_BC,), jnp.int32),
            pltpu.SemaphoreType.DMA,
            pltpu.SemaphoreType.DMA,
            pltpu.SemaphoreType.DMA,
            pltpu.SemaphoreType.DMA,
            pltpu.SemaphoreType.DMA,
            pltpu.SemaphoreType.DMA,
        ],
    )(nodes, Wn_n, Wn_sr, W_sr, W_ee, SR_flat, edges_flat, s2)


def kernel(nodes, edges, receivers, senders, W_node, W_edge):
    Wn_n = W_node[:, :DN]
    Wn_sr = W_node[:, DN:DN + 2 * DE]
    W_ee = W_edge[:, :DE]
    W_sr = jnp.concatenate([W_edge[:, DE:DE + DN], W_edge[:, DE + DN:]], axis=0)
    sr_idx = jnp.concatenate([senders, receivers])

    counts = _count_kernel(senders, receivers)
    s2, r2 = _rewrite_kernel(senders, receivers, counts)

    edges_flat = edges.reshape(-1)
    for _ in range(2):
        SR_flat = _segsum_kernel(edges_flat, sr_idx)
        nodes, AB_flat, C2_flat = _m13(
            nodes, SR_flat, edges_flat, s2, Wn_n, Wn_sr, W_sr, W_ee)
        edges_flat = _edge_kernel(C2_flat, AB_flat, s2, r2)
    return nodes, edges_flat.reshape(DE, E), receivers, senders


# speedup vs baseline: 78.2908x; 1.1787x over previous
import functools

import jax
import jax.numpy as jnp
from jax import lax
from jax.experimental import pallas as pl
from jax.experimental.pallas import tpu as pltpu
from jax.experimental.pallas import tpu_sc as plsc

NC = 2
NS = 16
NW = NC * NS
L = 16

DN = 128
DE = 16
N = 10000
E = 320000
K = E - N
NP = 10112

MESH = dict(
    mesh=plsc.VectorSubcoreMesh(
        core_axis_name="c", subcore_axis_name="s", num_cores=NC, num_subcores=NS),
    compiler_params=pltpu.CompilerParams(needs_layout_passes=False),
)


def _wid():
    return lax.axis_index("s") * NC + lax.axis_index("c")


_CHM = E // NW


@functools.partial(
    pl.kernel,
    out_type=(
        jax.ShapeDtypeStruct((NW * L,), jnp.int32),
        jax.ShapeDtypeStruct((2 * E,), jnp.int32),
    ),
    scratch_types=[
        pltpu.VMEM((_CHM,), jnp.int32),
        pltpu.VMEM((_CHM,), jnp.int32),
        pltpu.VMEM((L,), jnp.int32),
    ],
    **MESH,
)
def _count_kernel(s_hbm, r_hbm, out_hbm, sr_hbm, sbuf, rbuf, cbuf):
    w = _wid()
    base = w * _CHM
    pltpu.sync_copy(s_hbm.at[pl.ds(base, _CHM)], sbuf)
    pltpu.sync_copy(r_hbm.at[pl.ds(base, _CHM)], rbuf)
    pltpu.sync_copy(sbuf, sr_hbm.at[pl.ds(base, _CHM)])
    pltpu.sync_copy(rbuf, sr_hbm.at[pl.ds(E + base, _CHM)])

    def body(j, cnt):
        sv = sbuf[pl.ds(j * L, L)]
        rv = rbuf[pl.ds(j * L, L)]
        return cnt + jnp.where(sv != rv, 1, 0).astype(jnp.int32)

    cnt = lax.fori_loop(0, _CHM // L, body, jnp.zeros((L,), jnp.int32))
    cbuf[...] = jnp.broadcast_to(jnp.sum(cnt), (L,))
    pltpu.sync_copy(cbuf, out_hbm.at[pl.ds(w * L, L)])


@functools.partial(
    pl.kernel,
    out_type=(
        jax.ShapeDtypeStruct((E,), jnp.int32),
        jax.ShapeDtypeStruct((E,), jnp.int32),
    ),
    scratch_types=[
        pltpu.VMEM((_CHM,), jnp.int32),
        pltpu.VMEM((_CHM,), jnp.int32),
        pltpu.VMEM((_CHM,), jnp.int32),
        pltpu.VMEM((_CHM,), jnp.int32),
        pltpu.VMEM((NW * L,), jnp.int32),
    ],
    **MESH,
)
def _rewrite_kernel(s_hbm, r_hbm, cnt_hbm, s2_hbm, r2_hbm, sbuf, rbuf, s2buf, r2buf, cntbuf):
    w = _wid()
    base = w * _CHM
    pltpu.sync_copy(s_hbm.at[pl.ds(base, _CHM)], sbuf)
    pltpu.sync_copy(r_hbm.at[pl.ds(base, _CHM)], rbuf)
    pltpu.sync_copy(cnt_hbm, cntbuf)

    off = jnp.zeros((L,), jnp.int32)
    tot = jnp.zeros((L,), jnp.int32)
    for k in range(NW):
        ck = cntbuf[pl.ds(k * L, L)]
        off = off + jnp.where(jnp.broadcast_to(k < w, (L,)), ck, 0)
        tot = tot + ck
    lane = lax.iota(jnp.int32, L)
    pad_force = jnp.broadcast_to((w == 0) & (jnp.sum(tot) // L < K), (L,)) & (lane == 0)

    def body(j, running):
        sv = sbuf[pl.ds(j * L, L)]
        rv = rbuf[pl.ds(j * L, L)]
        m = sv != rv
        mi = jnp.where(m, 1, 0).astype(jnp.int32)
        incl = plsc.cumsum(mi)
        rank = incl + running
        upd = m & (rank <= K)
        upd = upd | (pad_force & jnp.broadcast_to(j == 0, (L,)))
        s2buf[pl.ds(j * L, L)] = jnp.where(upd, sv, N)
        r2buf[pl.ds(j * L, L)] = jnp.where(upd, rv, N)
        return running + jnp.max(incl)

    lax.fori_loop(0, _CHM // L, body, jnp.sum(off) // L)
    pltpu.sync_copy(s2buf, s2_hbm.at[pl.ds(base, _CHM)])
    pltpu.sync_copy(r2buf, r2_hbm.at[pl.ds(base, _CHM)])


_CH1 = 16000


@functools.partial(
    pl.kernel,
    out_type=jax.ShapeDtypeStruct((2 * DE * NP,), jnp.float32),
    scratch_types=[
        pltpu.VMEM((N,), jnp.float32),
        pltpu.VMEM((_CH1,), jnp.float32),
        pltpu.VMEM((_CH1,), jnp.float32),
        pltpu.VMEM((_CH1,), jnp.int32),
        pltpu.VMEM((_CH1,), jnp.int32),
        pltpu.SemaphoreType.DMA,
        pltpu.SemaphoreType.DMA,
    ],
    **MESH,
)
def _segsum_kernel(edges_hbm, sridx_hbm, out_hbm, acc, vbuf0, vbuf1, ibuf0, ibuf1,
                   sem0, sem1):
    w = _wid()
    t = w // DE
    f = w % DE
    nch = E // _CH1

    def start(c, vb, ib, sem):
        base = c * _CH1
        pltpu.async_copy(edges_hbm.at[pl.ds(f * E + base, _CH1)], vb, sem)
        pltpu.async_copy(sridx_hbm.at[pl.ds(t * E + base, _CH1)], ib, sem)

    def drain(vb, ib, sem):
        pltpu.make_async_copy(edges_hbm.at[pl.ds(f * E, _CH1)], vb, sem).wait()
        pltpu.make_async_copy(sridx_hbm.at[pl.ds(t * E, _CH1)], ib, sem).wait()

    def process(vb, ib):
        @plsc.parallel_loop(0, _CH1, L, unroll=8)
        def _scat(j):
            iv = ib[pl.ds(j, L)]
            vv = vb[pl.ds(j, L)]
            plsc.addupdate_scatter(acc, [iv], vv)

    start(0, vbuf0, ibuf0, sem0)

    @plsc.parallel_loop(0, N, L, unroll=8)
    def _zero(i):
        acc[pl.ds(i, L)] = jnp.zeros((L,), jnp.float32)

    def pair(p, _):
        c0 = 2 * p
        start(c0 + 1, vbuf1, ibuf1, sem1)
        drain(vbuf0, ibuf0, sem0)
        process(vbuf0, ibuf0)

        @pl.when(c0 + 2 < nch)
        def _():
            start(c0 + 2, vbuf0, ibuf0, sem0)

        drain(vbuf1, ibuf1, sem1)
        process(vbuf1, ibuf1)
        return 0

    lax.fori_loop(0, nch // 2, pair, 0)
    pltpu.sync_copy(acc, out_hbm.at[pl.ds(w * NP, N)])


_CH2 = 8000
_HALF = E // 2


@functools.partial(
    pl.kernel,
    out_type=jax.ShapeDtypeStruct((DE * E,), jnp.float32),
    scratch_types=[
        pltpu.VMEM((N + L,), jnp.float32),
        pltpu.VMEM((N + L,), jnp.float32),
        pltpu.VMEM((_CH2,), jnp.float32),
        pltpu.VMEM((_CH2,), jnp.float32),
        pltpu.VMEM((_CH2,), jnp.int32),
        pltpu.VMEM((_CH2,), jnp.int32),
        pltpu.VMEM((_CH2,), jnp.int32),
        pltpu.VMEM((_CH2,), jnp.int32),
        pltpu.VMEM((_CH2,), jnp.float32),
        pltpu.VMEM((_CH2,), jnp.float32),
        pltpu.SemaphoreType.DMA,
        pltpu.SemaphoreType.DMA,
        pltpu.SemaphoreType.DMA,
        pltpu.SemaphoreType.DMA,
    ],
    **MESH,
)
def _edge_kernel(c2_hbm, ab_hbm, s2_hbm, r2_hbm, out_hbm, arow, brow,
                 cbuf0, cbuf1, sbuf0, sbuf1, rbuf0, rbuf1, obuf0, obuf1,
                 sem0, sem1, osem0, osem1):
    w = _wid()
    h = w // DE
    f = w % DE
    nch = _HALF // _CH2

    def start(c, cb, sb, rb, sem):
        base = h * _HALF + c * _CH2
        pltpu.async_copy(c2_hbm.at[pl.ds(f * E + base, _CH2)], cb, sem)
        pltpu.async_copy(s2_hbm.at[pl.ds(base, _CH2)], sb, sem)
        pltpu.async_copy(r2_hbm.at[pl.ds(base, _CH2)], rb, sem)

    def drain(cb, sb, rb, sem):
        pltpu.make_async_copy(c2_hbm.at[pl.ds(0, _CH2)], cb, sem).wait()
        pltpu.make_async_copy(s2_hbm.at[pl.ds(0, _CH2)], sb, sem).wait()
        pltpu.make_async_copy(r2_hbm.at[pl.ds(0, _CH2)], rb, sem).wait()

    def process(c, cb, sb, rb, ob, osem):
        @plsc.parallel_loop(0, _CH2, L, unroll=8)
        def _gat(j):
            sv = sb[pl.ds(j, L)]
            rv = rb[pl.ds(j, L)]
            cv = cb[pl.ds(j, L)]
            av = plsc.load_gather(arow, [sv])
            bv = plsc.load_gather(brow, [rv])
            ob[pl.ds(j, L)] = cv + av + bv

        base = h * _HALF + c * _CH2
        pltpu.async_copy(ob, out_hbm.at[pl.ds(f * E + base, _CH2)], osem)

    def owait(ob, osem):
        pltpu.make_async_copy(ob, out_hbm.at[pl.ds(0, _CH2)], osem).wait()

    start(0, cbuf0, sbuf0, rbuf0, sem0)
    pltpu.sync_copy(ab_hbm.at[pl.ds(f * NP, N)], arow.at[pl.ds(0, N)])
    pltpu.sync_copy(ab_hbm.at[pl.ds((DE + f) * NP, N)], brow.at[pl.ds(0, N)])
    arow[pl.ds(N, L)] = jnp.zeros((L,), jnp.float32)
    brow[pl.ds(N, L)] = jnp.zeros((L,), jnp.float32)

    def pair(p, _):
        c0 = 2 * p
        start(c0 + 1, cbuf1, sbuf1, rbuf1, sem1)
        drain(cbuf0, sbuf0, rbuf0, sem0)

        @pl.when(c0 >= 2)
        def _():
            owait(obuf0, osem0)

        process(c0, cbuf0, sbuf0, rbuf0, obuf0, osem0)

        @pl.when(c0 + 2 < nch)
        def _():
            start(c0 + 2, cbuf0, sbuf0, rbuf0, sem0)

        drain(cbuf1, sbuf1, rbuf1, sem1)

        @pl.when(c0 >= 2)
        def _():
            owait(obuf1, osem1)

        process(c0 + 1, cbuf1, sbuf1, rbuf1, obuf1, osem1)
        return 0

    lax.fori_loop(0, nch // 2, pair, 0)
    owait(obuf0, osem0)
    owait(obuf1, osem1)


def _m1_body(nodes_ref, wnn_ref, wnsr_ref, wsr_ref, sr_hbm,
             nodes_out, ab_hbm, sr_v, ab_v, sem_sr, sem_ab):
    dot = functools.partial(
        jnp.dot, precision=lax.Precision.HIGHEST, preferred_element_type=jnp.float32
    )
    for rr in range(2 * DE):
        pltpu.async_copy(sr_hbm.at[pl.ds(rr * NP, NP)], sr_v.at[rr], sem_sr)
    for rr in range(2 * DE):
        pltpu.make_async_copy(sr_hbm.at[pl.ds(0, NP)], sr_v.at[rr], sem_sr).wait()
    nn = (dot(wnn_ref[...], nodes_ref[...])
          + dot(wnsr_ref[...], sr_v[...])[:, :N])
    nodes_out[...] = nn
    ab_v[:, pl.ds(0, N)] = dot(wsr_ref[...], nn)
    for rr in range(2 * DE):
        pltpu.async_copy(ab_v.at[rr], ab_hbm.at[pl.ds(rr * NP, NP)], sem_ab)
    for rr in range(2 * DE):
        pltpu.make_async_copy(ab_v.at[rr], ab_hbm.at[pl.ds(0, NP)], sem_ab).wait()


def _m1(nodes, SR_flat, Wn_n, Wn_sr, W_sr):
    hbm = pl.BlockSpec(memory_space=pltpu.MemorySpace.HBM)
    vmem = pl.BlockSpec(memory_space=pltpu.MemorySpace.VMEM)
    return pl.pallas_call(
        _m1_body,
        in_specs=[vmem, vmem, vmem, vmem, hbm],
        out_specs=(vmem, hbm),
        out_shape=(
            jax.ShapeDtypeStruct((DN, N), jnp.float32),
            jax.ShapeDtypeStruct((2 * DE * NP,), jnp.float32),
        ),
        scratch_shapes=[
            pltpu.VMEM((2 * DE, NP), jnp.float32),
            pltpu.VMEM((2 * DE, NP), jnp.float32),
            pltpu.SemaphoreType.DMA,
            pltpu.SemaphoreType.DMA,
        ],
    )(nodes, Wn_n, Wn_sr, W_sr, SR_flat)


_BC = 32000
_NB = E // _BC


def _m3_body(wee_ref, e_hbm, s2_hbm, c2_hbm,
             e0, e1, c0, c1, s0, s1, semi0, semi1, semo0, semo1):
    dot = functools.partial(
        jnp.dot, precision=lax.Precision.HIGHEST, preferred_element_type=jnp.float32
    )

    def start_in(c, eb, sb, sem):
        base = c * _BC
        for ff in range(DE):
            pltpu.async_copy(e_hbm.at[pl.ds(ff * E + base, _BC)], eb.at[ff], sem)
        pltpu.async_copy(s2_hbm.at[pl.ds(base, _BC)], sb, sem)

    def drain_in(eb, sb, sem):
        for ff in range(DE):
            pltpu.make_async_copy(e_hbm.at[pl.ds(0, _BC)], eb.at[ff], sem).wait()
        pltpu.make_async_copy(s2_hbm.at[pl.ds(0, _BC)], sb, sem).wait()

    def start_out(c, cb, sem):
        base = c * _BC
        for ff in range(DE):
            pltpu.async_copy(cb.at[ff], c2_hbm.at[pl.ds(ff * E + base, _BC)], sem)

    def drain_out(cb, sem):
        for ff in range(DE):
            pltpu.make_async_copy(cb.at[ff], c2_hbm.at[pl.ds(0, _BC)], sem).wait()

    start_in(0, e0, s0, semi0)
    start_in(1, e1, s1, semi1)
    for c in range(_NB):
        eb, sb, cb, semi, semo = ((e0, s0, c0, semi0, semo0) if c % 2 == 0
                                  else (e1, s1, c1, semi1, semo1))
        drain_in(eb, sb, semi)
        if c >= 2:
            drain_out(cb, semo)
        e = eb[...]
        cm = dot(wee_ref[...], e)
        u = (sb[...] != N).reshape(1, _BC)
        cb[...] = jnp.where(u, cm, e)
        start_out(c, cb, semo)
        if c + 2 < _NB:
            start_in(c + 2, eb, sb, semi)

    drain_out(c0 if _NB % 2 == 0 else c1, semo0 if _NB % 2 == 0 else semo1)
    drain_out(c1 if _NB % 2 == 0 else c0, semo1 if _NB % 2 == 0 else semo0)


def _m3(edges_flat, s2, W_ee):
    hbm = pl.BlockSpec(memory_space=pltpu.MemorySpace.HBM)
    vmem = pl.BlockSpec(memory_space=pltpu.MemorySpace.VMEM)
    return pl.pallas_call(
        _m3_body,
        in_specs=[vmem, hbm, hbm],
        out_specs=hbm,
        out_shape=jax.ShapeDtypeStruct((DE * E,), jnp.float32),
        scratch_shapes=[
            pltpu.VMEM((DE, _BC), jnp.float32),
            pltpu.VMEM((DE, _BC), jnp.float32),
            pltpu.VMEM((DE, _BC), jnp.float32),
            pltpu.VMEM((DE, _BC), jnp.float32),
            pltpu.VMEM((_BC,), jnp.int32),
            pltpu.VMEM((_BC,), jnp.int32),
            pltpu.SemaphoreType.DMA,
            pltpu.SemaphoreType.DMA,
            pltpu.SemaphoreType.DMA,
            pltpu.SemaphoreType.DMA,
        ],
    )(W_ee, edges_flat, s2)


def kernel(nodes, edges, receivers, senders, W_node, W_edge):
    Wn_n = W_node[:, :DN]
    Wn_sr = W_node[:, DN:DN + 2 * DE]
    W_ee = W_edge[:, :DE]
    W_sr = jnp.concatenate([W_edge[:, DE:DE + DN], W_edge[:, DE + DN:]], axis=0)
    counts, sr_idx = _count_kernel(senders, receivers)
    s2, r2 = _rewrite_kernel(senders, receivers, counts)

    edges_flat = edges.reshape(-1)
    for _ in range(2):
        SR_flat = _segsum_kernel(edges_flat, sr_idx)
        C2_flat = _m3(edges_flat, s2, W_ee)
        nodes, AB_flat = _m1(nodes, SR_flat, Wn_n, Wn_sr, W_sr)
        edges_flat = _edge_kernel(C2_flat, AB_flat, s2, r2)
    return nodes, edges_flat.reshape(DE, E), receivers, senders


# speedup vs baseline: 82.0699x; 1.0483x over previous
import functools

import jax
import jax.numpy as jnp
from jax import lax
from jax.experimental import pallas as pl
from jax.experimental.pallas import tpu as pltpu
from jax.experimental.pallas import tpu_sc as plsc

NC = 2
NS = 16
NW = NC * NS
L = 16

DN = 128
DE = 16
N = 10000
E = 320000
K = E - N
NP = 10112

MESH = dict(
    mesh=plsc.VectorSubcoreMesh(
        core_axis_name="c", subcore_axis_name="s", num_cores=NC, num_subcores=NS),
    compiler_params=pltpu.CompilerParams(needs_layout_passes=False),
)


def _wid():
    return lax.axis_index("s") * NC + lax.axis_index("c")


_CHM = E // NW


@functools.partial(
    pl.kernel,
    out_type=jax.ShapeDtypeStruct((NW * L,), jnp.int32),
    scratch_types=[
        pltpu.VMEM((_CHM,), jnp.int32),
        pltpu.VMEM((_CHM,), jnp.int32),
        pltpu.VMEM((L,), jnp.int32),
    ],
    **MESH,
)
def _count_kernel(s_hbm, r_hbm, out_hbm, sbuf, rbuf, cbuf):
    w = _wid()
    base = w * _CHM
    pltpu.sync_copy(s_hbm.at[pl.ds(base, _CHM)], sbuf)
    pltpu.sync_copy(r_hbm.at[pl.ds(base, _CHM)], rbuf)

    def body(j, cnt):
        sv = sbuf[pl.ds(j * L, L)]
        rv = rbuf[pl.ds(j * L, L)]
        return cnt + jnp.where(sv != rv, 1, 0).astype(jnp.int32)

    cnt = lax.fori_loop(0, _CHM // L, body, jnp.zeros((L,), jnp.int32))
    cbuf[...] = jnp.broadcast_to(jnp.sum(cnt), (L,))
    pltpu.sync_copy(cbuf, out_hbm.at[pl.ds(w * L, L)])


@functools.partial(
    pl.kernel,
    out_type=jax.ShapeDtypeStruct((E,), jnp.int32),
    scratch_types=[
        pltpu.VMEM((_CHM,), jnp.int32),
        pltpu.VMEM((_CHM,), jnp.int32),
        pltpu.VMEM((_CHM,), jnp.int32),
        pltpu.VMEM((NW * L,), jnp.int32),
    ],
    **MESH,
)
def _rewrite_kernel(s_hbm, r_hbm, cnt_hbm, sr2_hbm, sbuf, rbuf, sr2buf, cntbuf):
    w = _wid()
    base = w * _CHM
    pltpu.sync_copy(s_hbm.at[pl.ds(base, _CHM)], sbuf)
    pltpu.sync_copy(r_hbm.at[pl.ds(base, _CHM)], rbuf)
    pltpu.sync_copy(cnt_hbm, cntbuf)

    off = jnp.zeros((L,), jnp.int32)
    tot = jnp.zeros((L,), jnp.int32)
    for k in range(NW):
        ck = cntbuf[pl.ds(k * L, L)]
        off = off + jnp.where(jnp.broadcast_to(k < w, (L,)), ck, 0)
        tot = tot + ck
    lane = lax.iota(jnp.int32, L)
    pad_force = jnp.broadcast_to((w == 0) & (jnp.sum(tot) // L < K), (L,)) & (lane == 0)

    def body(j, running):
        sv = sbuf[pl.ds(j * L, L)]
        rv = rbuf[pl.ds(j * L, L)]
        m = sv != rv
        mi = jnp.where(m, 1, 0).astype(jnp.int32)
        incl = plsc.cumsum(mi)
        rank = incl + running
        upd = m & (rank <= K)
        upd = upd | (pad_force & jnp.broadcast_to(j == 0, (L,)))
        s2v = jnp.where(upd, sv, N)
        r2v = jnp.where(upd, rv, N)
        sr2buf[pl.ds(j * L, L)] = s2v | (r2v << 16)
        return running + jnp.max(incl)

    lax.fori_loop(0, _CHM // L, body, jnp.sum(off) // L)
    pltpu.sync_copy(sr2buf, sr2_hbm.at[pl.ds(base, _CHM)])


_CH1 = 8000
_HALF1 = E // 2


@functools.partial(
    pl.kernel,
    out_type=jax.ShapeDtypeStruct((4 * DE * NP,), jnp.float32),
    scratch_types=[
        pltpu.VMEM((N,), jnp.float32),
        pltpu.VMEM((N,), jnp.float32),
        pltpu.VMEM((_CH1,), jnp.float32),
        pltpu.VMEM((_CH1,), jnp.float32),
        pltpu.VMEM((_CH1,), jnp.int32),
        pltpu.VMEM((_CH1,), jnp.int32),
        pltpu.VMEM((_CH1,), jnp.int32),
        pltpu.VMEM((_CH1,), jnp.int32),
        pltpu.SemaphoreType.DMA,
        pltpu.SemaphoreType.DMA,
    ],
    **MESH,
)
def _segsum_kernel(edges_hbm, s_hbm, r_hbm, out_hbm, acc_s, acc_r,
                   vbuf0, vbuf1, sbuf0, sbuf1, rbuf0, rbuf1, sem0, sem1):
    w = _wid()
    h = w // DE
    f = w % DE
    nch = _HALF1 // _CH1

    def start(c, vb, sb, rb, sem):
        base = h * _HALF1 + c * _CH1
        pltpu.async_copy(edges_hbm.at[pl.ds(f * E + base, _CH1)], vb, sem)
        pltpu.async_copy(s_hbm.at[pl.ds(base, _CH1)], sb, sem)
        pltpu.async_copy(r_hbm.at[pl.ds(base, _CH1)], rb, sem)

    def drain(vb, sb, rb, sem):
        pltpu.make_async_copy(edges_hbm.at[pl.ds(f * E, _CH1)], vb, sem).wait()
        pltpu.make_async_copy(s_hbm.at[pl.ds(0, _CH1)], sb, sem).wait()
        pltpu.make_async_copy(r_hbm.at[pl.ds(0, _CH1)], rb, sem).wait()

    def process(vb, sb, rb):
        @plsc.parallel_loop(0, _CH1, L, unroll=8)
        def _scat(j):
            vv = vb[pl.ds(j, L)]
            plsc.addupdate_scatter(acc_s, [sb[pl.ds(j, L)]], vv)
            plsc.addupdate_scatter(acc_r, [rb[pl.ds(j, L)]], vv)

    start(0, vbuf0, sbuf0, rbuf0, sem0)

    @plsc.parallel_loop(0, N, L, unroll=8)
    def _zero(i):
        acc_s[pl.ds(i, L)] = jnp.zeros((L,), jnp.float32)
        acc_r[pl.ds(i, L)] = jnp.zeros((L,), jnp.float32)

    def pair(p, _):
        c0 = 2 * p
        start(c0 + 1, vbuf1, sbuf1, rbuf1, sem1)
        drain(vbuf0, sbuf0, rbuf0, sem0)
        process(vbuf0, sbuf0, rbuf0)

        @pl.when(c0 + 2 < nch)
        def _():
            start(c0 + 2, vbuf0, sbuf0, rbuf0, sem0)

        drain(vbuf1, sbuf1, rbuf1, sem1)
        process(vbuf1, sbuf1, rbuf1)
        return 0

    lax.fori_loop(0, nch // 2, pair, 0)
    pltpu.sync_copy(acc_s, out_hbm.at[pl.ds((h * 2 * DE + f) * NP, N)])
    pltpu.sync_copy(acc_r, out_hbm.at[pl.ds((h * 2 * DE + DE + f) * NP, N)])


_CH2 = 8000
_HALF = E // 2


@functools.partial(
    pl.kernel,
    out_type=jax.ShapeDtypeStruct((DE * E,), jnp.float32),
    scratch_types=[
        pltpu.VMEM((N + L,), jnp.float32),
        pltpu.VMEM((N + L,), jnp.float32),
        pltpu.VMEM((_CH2,), jnp.float32),
        pltpu.VMEM((_CH2,), jnp.float32),
        pltpu.VMEM((_CH2,), jnp.int32),
        pltpu.VMEM((_CH2,), jnp.int32),
        pltpu.VMEM((_CH2,), jnp.float32),
        pltpu.VMEM((_CH2,), jnp.float32),
        pltpu.SemaphoreType.DMA,
        pltpu.SemaphoreType.DMA,
        pltpu.SemaphoreType.DMA,
        pltpu.SemaphoreType.DMA,
    ],
    **MESH,
)
def _edge_kernel(c2_hbm, ab_hbm, sr2_hbm, out_hbm, arow, brow,
                 cbuf0, cbuf1, sbuf0, sbuf1, obuf0, obuf1,
                 sem0, sem1, osem0, osem1):
    w = _wid()
    h = w // DE
    f = w % DE
    nch = _HALF // _CH2

    def start(c, cb, sb, sem):
        base = h * _HALF + c * _CH2
        pltpu.async_copy(c2_hbm.at[pl.ds(f * E + base, _CH2)], cb, sem)
        pltpu.async_copy(sr2_hbm.at[pl.ds(base, _CH2)], sb, sem)

    def drain(cb, sb, sem):
        pltpu.make_async_copy(c2_hbm.at[pl.ds(0, _CH2)], cb, sem).wait()
        pltpu.make_async_copy(sr2_hbm.at[pl.ds(0, _CH2)], sb, sem).wait()

    def process(c, cb, sb, ob, osem):
        @plsc.parallel_loop(0, _CH2, L, unroll=8)
        def _gat(j):
            pk = sb[pl.ds(j, L)]
            sv = pk & 0xFFFF
            rv = lax.shift_right_logical(pk, 16)
            cv = cb[pl.ds(j, L)]
            av = plsc.load_gather(arow, [sv])
            bv = plsc.load_gather(brow, [rv])
            ob[pl.ds(j, L)] = cv + av + bv

        base = h * _HALF + c * _CH2
        pltpu.async_copy(ob, out_hbm.at[pl.ds(f * E + base, _CH2)], osem)

    def owait(ob, osem):
        pltpu.make_async_copy(ob, out_hbm.at[pl.ds(0, _CH2)], osem).wait()

    start(0, cbuf0, sbuf0, sem0)
    pltpu.sync_copy(ab_hbm.at[pl.ds(f * NP, N)], arow.at[pl.ds(0, N)])
    pltpu.sync_copy(ab_hbm.at[pl.ds((DE + f) * NP, N)], brow.at[pl.ds(0, N)])
    arow[pl.ds(N, L)] = jnp.zeros((L,), jnp.float32)
    brow[pl.ds(N, L)] = jnp.zeros((L,), jnp.float32)

    def pair(p, _):
        c0 = 2 * p
        start(c0 + 1, cbuf1, sbuf1, sem1)
        drain(cbuf0, sbuf0, sem0)

        @pl.when(c0 >= 2)
        def _():
            owait(obuf0, osem0)

        process(c0, cbuf0, sbuf0, obuf0, osem0)

        @pl.when(c0 + 2 < nch)
        def _():
            start(c0 + 2, cbuf0, sbuf0, sem0)

        drain(cbuf1, sbuf1, sem1)

        @pl.when(c0 >= 2)
        def _():
            owait(obuf1, osem1)

        process(c0 + 1, cbuf1, sbuf1, obuf1, osem1)
        return 0

    lax.fori_loop(0, nch // 2, pair, 0)
    owait(obuf0, osem0)
    owait(obuf1, osem1)


def _m1_body(nodes_ref, wnn_ref, wnsr_ref, wsr_ref, sr_hbm,
             nodes_out, ab_hbm, sr_v, ab_v, sem_sr, sem_ab):
    dot = functools.partial(
        jnp.dot, precision=lax.Precision.HIGHEST, preferred_element_type=jnp.float32
    )
    for rr in range(4 * DE):
        pltpu.async_copy(sr_hbm.at[pl.ds(rr * NP, NP)], sr_v.at[rr], sem_sr)
    for rr in range(4 * DE):
        pltpu.make_async_copy(sr_hbm.at[pl.ds(0, NP)], sr_v.at[rr], sem_sr).wait()
    srsum = sr_v[0:2 * DE, :] + sr_v[2 * DE:4 * DE, :]
    nn = (dot(wnn_ref[...], nodes_ref[...])
          + dot(wnsr_ref[...], srsum)[:, :N])
    nodes_out[...] = nn
    ab_v[:, pl.ds(0, N)] = dot(wsr_ref[...], nn)
    for rr in range(2 * DE):
        pltpu.async_copy(ab_v.at[rr], ab_hbm.at[pl.ds(rr * NP, NP)], sem_ab)
    for rr in range(2 * DE):
        pltpu.make_async_copy(ab_v.at[rr], ab_hbm.at[pl.ds(0, NP)], sem_ab).wait()


def _m1(nodes, SR_flat, Wn_n, Wn_sr, W_sr):
    hbm = pl.BlockSpec(memory_space=pltpu.MemorySpace.HBM)
    vmem = pl.BlockSpec(memory_space=pltpu.MemorySpace.VMEM)
    return pl.pallas_call(
        _m1_body,
        in_specs=[vmem, vmem, vmem, vmem, hbm],
        out_specs=(vmem, hbm),
        out_shape=(
            jax.ShapeDtypeStruct((DN, N), jnp.float32),
            jax.ShapeDtypeStruct((2 * DE * NP,), jnp.float32),
        ),
        scratch_shapes=[
            pltpu.VMEM((4 * DE, NP), jnp.float32),
            pltpu.VMEM((2 * DE, NP), jnp.float32),
            pltpu.SemaphoreType.DMA,
            pltpu.SemaphoreType.DMA,
        ],
    )(nodes, Wn_n, Wn_sr, W_sr, SR_flat)


_BC = 32000
_NB = E // _BC


def _m3_body(wee_ref, e_hbm, s2_hbm, c2_hbm,
             e0, e1, c0, c1, s0, s1, semi0, semi1, semo0, semo1):
    dot = functools.partial(
        jnp.dot, precision=lax.Precision.HIGHEST, preferred_element_type=jnp.float32
    )

    def start_in(c, eb, sb, sem):
        base = c * _BC
        for ff in range(DE):
            pltpu.async_copy(e_hbm.at[pl.ds(ff * E + base, _BC)], eb.at[ff], sem)
        pltpu.async_copy(s2_hbm.at[pl.ds(base, _BC)], sb, sem)

    def drain_in(eb, sb, sem):
        for ff in range(DE):
            pltpu.make_async_copy(e_hbm.at[pl.ds(0, _BC)], eb.at[ff], sem).wait()
        pltpu.make_async_copy(s2_hbm.at[pl.ds(0, _BC)], sb, sem).wait()

    def start_out(c, cb, sem):
        base = c * _BC
        for ff in range(DE):
            pltpu.async_copy(cb.at[ff], c2_hbm.at[pl.ds(ff * E + base, _BC)], sem)

    def drain_out(cb, sem):
        for ff in range(DE):
            pltpu.make_async_copy(cb.at[ff], c2_hbm.at[pl.ds(0, _BC)], sem).wait()

    start_in(0, e0, s0, semi0)
    start_in(1, e1, s1, semi1)
    for c in range(_NB):
        eb, sb, cb, semi, semo = ((e0, s0, c0, semi0, semo0) if c % 2 == 0
                                  else (e1, s1, c1, semi1, semo1))
        drain_in(eb, sb, semi)
        if c >= 2:
            drain_out(cb, semo)
        e = eb[...]
        cm = dot(wee_ref[...], e)
        u = ((sb[...] & 0xFFFF) != N).reshape(1, _BC)
        cb[...] = jnp.where(u, cm, e)
        start_out(c, cb, semo)
        if c + 2 < _NB:
            start_in(c + 2, eb, sb, semi)

    drain_out(c0 if _NB % 2 == 0 else c1, semo0 if _NB % 2 == 0 else semo1)
    drain_out(c1 if _NB % 2 == 0 else c0, semo1 if _NB % 2 == 0 else semo0)


def _m3(edges_flat, s2, W_ee):
    hbm = pl.BlockSpec(memory_space=pltpu.MemorySpace.HBM)
    vmem = pl.BlockSpec(memory_space=pltpu.MemorySpace.VMEM)
    return pl.pallas_call(
        _m3_body,
        in_specs=[vmem, hbm, hbm],
        out_specs=hbm,
        out_shape=jax.ShapeDtypeStruct((DE * E,), jnp.float32),
        scratch_shapes=[
            pltpu.VMEM((DE, _BC), jnp.float32),
            pltpu.VMEM((DE, _BC), jnp.float32),
            pltpu.VMEM((DE, _BC), jnp.float32),
            pltpu.VMEM((DE, _BC), jnp.float32),
            pltpu.VMEM((_BC,), jnp.int32),
            pltpu.VMEM((_BC,), jnp.int32),
            pltpu.SemaphoreType.DMA,
            pltpu.SemaphoreType.DMA,
            pltpu.SemaphoreType.DMA,
            pltpu.SemaphoreType.DMA,
        ],
    )(W_ee, edges_flat, s2)


def kernel(nodes, edges, receivers, senders, W_node, W_edge):
    Wn_n = W_node[:, :DN]
    Wn_sr = W_node[:, DN:DN + 2 * DE]
    W_ee = W_edge[:, :DE]
    W_sr = jnp.concatenate([W_edge[:, DE:DE + DN], W_edge[:, DE + DN:]], axis=0)
    counts = _count_kernel(senders, receivers)
    sr2 = _rewrite_kernel(senders, receivers, counts)

    edges_flat = edges.reshape(-1)
    for _ in range(2):
        SR_flat = _segsum_kernel(edges_flat, senders, receivers)
        C2_flat = _m3(edges_flat, sr2, W_ee)
        nodes, AB_flat = _m1(nodes, SR_flat, Wn_n, Wn_sr, W_sr)
        edges_flat = _edge_kernel(C2_flat, AB_flat, sr2)
    return nodes, edges_flat.reshape(DE, E), receivers, senders


# speedup vs baseline: 82.0777x; 1.0001x over previous
import functools

import jax
import jax.numpy as jnp
from jax import lax
from jax.experimental import pallas as pl
from jax.experimental.pallas import tpu as pltpu
from jax.experimental.pallas import tpu_sc as plsc

NC = 2
NS = 16
NW = NC * NS
L = 16

DN = 128
DE = 16
N = 10000
E = 320000
K = E - N
NP = 10112

MESH = dict(
    mesh=plsc.VectorSubcoreMesh(
        core_axis_name="c", subcore_axis_name="s", num_cores=NC, num_subcores=NS),
    compiler_params=pltpu.CompilerParams(needs_layout_passes=False),
)


def _wid():
    return lax.axis_index("s") * NC + lax.axis_index("c")


_CHM = E // NW


@functools.partial(
    pl.kernel,
    out_type=jax.ShapeDtypeStruct((NW * L,), jnp.int32),
    scratch_types=[
        pltpu.VMEM((_CHM,), jnp.int32),
        pltpu.VMEM((_CHM,), jnp.int32),
        pltpu.VMEM((L,), jnp.int32),
    ],
    **MESH,
)
def _count_kernel(s_hbm, r_hbm, out_hbm, sbuf, rbuf, cbuf):
    w = _wid()
    base = w * _CHM
    pltpu.sync_copy(s_hbm.at[pl.ds(base, _CHM)], sbuf)
    pltpu.sync_copy(r_hbm.at[pl.ds(base, _CHM)], rbuf)

    def body(j, cnt):
        sv = sbuf[pl.ds(j * L, L)]
        rv = rbuf[pl.ds(j * L, L)]
        return cnt + jnp.where(sv != rv, 1, 0).astype(jnp.int32)

    cnt = lax.fori_loop(0, _CHM // L, body, jnp.zeros((L,), jnp.int32))
    cbuf[...] = jnp.broadcast_to(jnp.sum(cnt), (L,))
    pltpu.sync_copy(cbuf, out_hbm.at[pl.ds(w * L, L)])


@functools.partial(
    pl.kernel,
    out_type=jax.ShapeDtypeStruct((E,), jnp.int32),
    scratch_types=[
        pltpu.VMEM((_CHM,), jnp.int32),
        pltpu.VMEM((_CHM,), jnp.int32),
        pltpu.VMEM((_CHM,), jnp.int32),
        pltpu.VMEM((NW * L,), jnp.int32),
    ],
    **MESH,
)
def _rewrite_kernel(s_hbm, r_hbm, cnt_hbm, sr2_hbm, sbuf, rbuf, sr2buf, cntbuf):
    w = _wid()
    base = w * _CHM
    pltpu.sync_copy(s_hbm.at[pl.ds(base, _CHM)], sbuf)
    pltpu.sync_copy(r_hbm.at[pl.ds(base, _CHM)], rbuf)
    pltpu.sync_copy(cnt_hbm, cntbuf)

    off = jnp.zeros((L,), jnp.int32)
    tot = jnp.zeros((L,), jnp.int32)
    for k in range(NW):
        ck = cntbuf[pl.ds(k * L, L)]
        off = off + jnp.where(jnp.broadcast_to(k < w, (L,)), ck, 0)
        tot = tot + ck
    lane = lax.iota(jnp.int32, L)
    pad_force = jnp.broadcast_to((w == 0) & (jnp.sum(tot) // L < K), (L,)) & (lane == 0)

    def body(j, running):
        sv = sbuf[pl.ds(j * L, L)]
        rv = rbuf[pl.ds(j * L, L)]
        m = sv != rv
        mi = jnp.where(m, 1, 0).astype(jnp.int32)
        incl = plsc.cumsum(mi)
        rank = incl + running
        upd = m & (rank <= K)
        upd = upd | (pad_force & jnp.broadcast_to(j == 0, (L,)))
        s2v = jnp.where(upd, sv, N)
        r2v = jnp.where(upd, rv, N)
        sr2buf[pl.ds(j * L, L)] = s2v | (r2v << 16)
        return running + jnp.max(incl)

    lax.fori_loop(0, _CHM // L, body, jnp.sum(off) // L)
    pltpu.sync_copy(sr2buf, sr2_hbm.at[pl.ds(base, _CHM)])


_CH1 = 8000
_HALF1 = E // 2


@functools.partial(
    pl.kernel,
    out_type=jax.ShapeDtypeStruct((4 * DE * NP,), jnp.float32),
    scratch_types=[
        pltpu.VMEM((N,), jnp.float32),
        pltpu.VMEM((N,), jnp.float32),
        pltpu.VMEM((_CH1,), jnp.float32),
        pltpu.VMEM((_CH1,), jnp.float32),
        pltpu.VMEM((_CH1,), jnp.int32),
        pltpu.VMEM((_CH1,), jnp.int32),
        pltpu.VMEM((_CH1,), jnp.int32),
        pltpu.VMEM((_CH1,), jnp.int32),
        pltpu.SemaphoreType.DMA,
        pltpu.SemaphoreType.DMA,
    ],
    **MESH,
)
def _segsum_kernel(edges_hbm, s_hbm, r_hbm, out_hbm, acc_s, acc_r,
                   vbuf0, vbuf1, sbuf0, sbuf1, rbuf0, rbuf1, sem0, sem1):
    w = _wid()
    h = w // DE
    f = w % DE
    nch = _HALF1 // _CH1

    def start(c, vb, sb, rb, sem):
        base = h * _HALF1 + c * _CH1
        pltpu.async_copy(edges_hbm.at[pl.ds(f * E + base, _CH1)], vb, sem)
        pltpu.async_copy(s_hbm.at[pl.ds(base, _CH1)], sb, sem)
        pltpu.async_copy(r_hbm.at[pl.ds(base, _CH1)], rb, sem)

    def drain(vb, sb, rb, sem):
        pltpu.make_async_copy(edges_hbm.at[pl.ds(f * E, _CH1)], vb, sem).wait()
        pltpu.make_async_copy(s_hbm.at[pl.ds(0, _CH1)], sb, sem).wait()
        pltpu.make_async_copy(r_hbm.at[pl.ds(0, _CH1)], rb, sem).wait()

    def process(vb, sb, rb):
        @plsc.parallel_loop(0, _CH1, L, unroll=8)
        def _scat(j):
            vv = vb[pl.ds(j, L)]
            plsc.addupdate_scatter(acc_s, [sb[pl.ds(j, L)]], vv)
            plsc.addupdate_scatter(acc_r, [rb[pl.ds(j, L)]], vv)

    start(0, vbuf0, sbuf0, rbuf0, sem0)

    @plsc.parallel_loop(0, N, L, unroll=8)
    def _zero(i):
        acc_s[pl.ds(i, L)] = jnp.zeros((L,), jnp.float32)
        acc_r[pl.ds(i, L)] = jnp.zeros((L,), jnp.float32)

    def pair(p, _):
        c0 = 2 * p
        start(c0 + 1, vbuf1, sbuf1, rbuf1, sem1)
        drain(vbuf0, sbuf0, rbuf0, sem0)
        process(vbuf0, sbuf0, rbuf0)

        @pl.when(c0 + 2 < nch)
        def _():
            start(c0 + 2, vbuf0, sbuf0, rbuf0, sem0)

        drain(vbuf1, sbuf1, rbuf1, sem1)
        process(vbuf1, sbuf1, rbuf1)
        return 0

    lax.fori_loop(0, nch // 2, pair, 0)
    pltpu.sync_copy(acc_s, out_hbm.at[pl.ds((h * 2 * DE + f) * NP, N)])
    pltpu.sync_copy(acc_r, out_hbm.at[pl.ds((h * 2 * DE + DE + f) * NP, N)])


_CH2 = 8000
_HALF = E // 2


@functools.partial(
    pl.kernel,
    out_type=jax.ShapeDtypeStruct((DE * E,), jnp.float32),
    scratch_types=[
        pltpu.VMEM((N + L,), jnp.float32),
        pltpu.VMEM((N + L,), jnp.float32),
        pltpu.VMEM((_CH2,), jnp.float32),
        pltpu.VMEM((_CH2,), jnp.float32),
        pltpu.VMEM((_CH2,), jnp.int32),
        pltpu.VMEM((_CH2,), jnp.int32),
        pltpu.VMEM((_CH2,), jnp.float32),
        pltpu.VMEM((_CH2,), jnp.float32),
        pltpu.SemaphoreType.DMA,
        pltpu.SemaphoreType.DMA,
        pltpu.SemaphoreType.DMA,
        pltpu.SemaphoreType.DMA,
    ],
    **MESH,
)
def _edge_kernel(c2_hbm, ab_hbm, sr2_hbm, out_hbm, arow, brow,
                 cbuf0, cbuf1, sbuf0, sbuf1, obuf0, obuf1,
                 sem0, sem1, osem0, osem1):
    w = _wid()
    h = w // DE
    f = w % DE
    nch = _HALF // _CH2

    def start(c, cb, sb, sem):
        base = h * _HALF + c * _CH2
        pltpu.async_copy(c2_hbm.at[pl.ds(f * E + base, _CH2)], cb, sem)
        pltpu.async_copy(sr2_hbm.at[pl.ds(base, _CH2)], sb, sem)

    def drain(cb, sb, sem):
        pltpu.make_async_copy(c2_hbm.at[pl.ds(0, _CH2)], cb, sem).wait()
        pltpu.make_async_copy(sr2_hbm.at[pl.ds(0, _CH2)], sb, sem).wait()

    def process(c, cb, sb, ob, osem):
        @plsc.parallel_loop(0, _CH2, L, unroll=8)
        def _gat(j):
            pk = sb[pl.ds(j, L)]
            sv = pk & 0xFFFF
            rv = lax.shift_right_logical(pk, 16)
            cv = cb[pl.ds(j, L)]
            av = plsc.load_gather(arow, [sv])
            bv = plsc.load_gather(brow, [rv])
            ob[pl.ds(j, L)] = cv + av + bv

        base = h * _HALF + c * _CH2
        pltpu.async_copy(ob, out_hbm.at[pl.ds(f * E + base, _CH2)], osem)

    def owait(ob, osem):
        pltpu.make_async_copy(ob, out_hbm.at[pl.ds(0, _CH2)], osem).wait()

    start(0, cbuf0, sbuf0, sem0)
    pltpu.sync_copy(ab_hbm.at[pl.ds(f * NP, N)], arow.at[pl.ds(0, N)])
    pltpu.sync_copy(ab_hbm.at[pl.ds((DE + f) * NP, N)], brow.at[pl.ds(0, N)])
    arow[pl.ds(N, L)] = jnp.zeros((L,), jnp.float32)
    brow[pl.ds(N, L)] = jnp.zeros((L,), jnp.float32)

    def pair(p, _):
        c0 = 2 * p
        start(c0 + 1, cbuf1, sbuf1, sem1)
        drain(cbuf0, sbuf0, sem0)

        @pl.when(c0 >= 2)
        def _():
            owait(obuf0, osem0)

        process(c0, cbuf0, sbuf0, obuf0, osem0)

        @pl.when(c0 + 2 < nch)
        def _():
            start(c0 + 2, cbuf0, sbuf0, sem0)

        drain(cbuf1, sbuf1, sem1)

        @pl.when(c0 >= 2)
        def _():
            owait(obuf1, osem1)

        process(c0 + 1, cbuf1, sbuf1, obuf1, osem1)
        return 0

    lax.fori_loop(0, nch // 2, pair, 0)
    owait(obuf0, osem0)
    owait(obuf1, osem1)


def _m1_body(nodes_ref, wnn_ref, wnsr_ref, wsr_ref, sr_hbm,
             nodes_out, ab_hbm, sr_v, ab_v, sem_sr, sem_ab):
    dot = functools.partial(
        jnp.dot, precision=lax.Precision.HIGHEST, preferred_element_type=jnp.float32
    )
    for rr in range(4 * DE):
        pltpu.async_copy(sr_hbm.at[pl.ds(rr * NP, NP)], sr_v.at[rr], sem_sr)
    for rr in range(4 * DE):
        pltpu.make_async_copy(sr_hbm.at[pl.ds(0, NP)], sr_v.at[rr], sem_sr).wait()
    srsum = sr_v[0:2 * DE, :] + sr_v[2 * DE:4 * DE, :]
    nn = (dot(wnn_ref[...], nodes_ref[...])
          + dot(wnsr_ref[...], srsum)[:, :N])
    nodes_out[...] = nn
    ab_v[:, pl.ds(0, N)] = dot(wsr_ref[...], nn)
    for rr in range(2 * DE):
        pltpu.async_copy(ab_v.at[rr], ab_hbm.at[pl.ds(rr * NP, NP)], sem_ab)
    for rr in range(2 * DE):
        pltpu.make_async_copy(ab_v.at[rr], ab_hbm.at[pl.ds(0, NP)], sem_ab).wait()


def _m1(nodes, SR_flat, Wn_n, Wn_sr, W_sr):
    hbm = pl.BlockSpec(memory_space=pltpu.MemorySpace.HBM)
    vmem = pl.BlockSpec(memory_space=pltpu.MemorySpace.VMEM)
    return pl.pallas_call(
        _m1_body,
        in_specs=[vmem, vmem, vmem, vmem, hbm],
        out_specs=(vmem, hbm),
        out_shape=(
            jax.ShapeDtypeStruct((DN, N), jnp.float32),
            jax.ShapeDtypeStruct((2 * DE * NP,), jnp.float32),
        ),
        scratch_shapes=[
            pltpu.VMEM((4 * DE, NP), jnp.float32),
            pltpu.VMEM((2 * DE, NP), jnp.float32),
            pltpu.SemaphoreType.DMA,
            pltpu.SemaphoreType.DMA,
        ],
    )(nodes, Wn_n, Wn_sr, W_sr, SR_flat)


_BC = 32000
_NB = E // _BC


def _m3_body(wee_ref, e_hbm, s2_hbm, c2_hbm,
             e0, e1, c0, c1, s0, s1, semi0, semi1, semo0, semo1):
    dot = functools.partial(
        jnp.dot, precision=lax.Precision.HIGHEST, preferred_element_type=jnp.float32
    )

    def start_in(c, eb, sb, sem):
        base = c * _BC
        for ff in range(DE):
            pltpu.async_copy(e_hbm.at[pl.ds(ff * E + base, _BC)], eb.at[ff], sem)
        pltpu.async_copy(s2_hbm.at[pl.ds(base, _BC)], sb, sem)

    def drain_in(eb, sb, sem):
        for ff in range(DE):
            pltpu.make_async_copy(e_hbm.at[pl.ds(0, _BC)], eb.at[ff], sem).wait()
        pltpu.make_async_copy(s2_hbm.at[pl.ds(0, _BC)], sb, sem).wait()

    def start_out(c, cb, sem):
        base = c * _BC
        for ff in range(DE):
            pltpu.async_copy(cb.at[ff], c2_hbm.at[pl.ds(ff * E + base, _BC)], sem)

    def drain_out(cb, sem):
        for ff in range(DE):
            pltpu.make_async_copy(cb.at[ff], c2_hbm.at[pl.ds(0, _BC)], sem).wait()

    start_in(0, e0, s0, semi0)
    start_in(1, e1, s1, semi1)
    for c in range(_NB):
        eb, sb, cb, semi, semo = ((e0, s0, c0, semi0, semo0) if c % 2 == 0
                                  else (e1, s1, c1, semi1, semo1))
        drain_in(eb, sb, semi)
        if c >= 2:
            drain_out(cb, semo)
        e = eb[...]
        cm = dot(wee_ref[...], e)
        u = ((sb[...] & 0xFFFF) != N).reshape(1, _BC)
        cb[...] = jnp.where(u, cm, e)
        start_out(c, cb, semo)
        if c + 2 < _NB:
            start_in(c + 2, eb, sb, semi)

    drain_out(c0 if _NB % 2 == 0 else c1, semo0 if _NB % 2 == 0 else semo1)
    drain_out(c1 if _NB % 2 == 0 else c0, semo1 if _NB % 2 == 0 else semo0)


def _m3(edges_flat, s2, W_ee):
    hbm = pl.BlockSpec(memory_space=pltpu.MemorySpace.HBM)
    vmem = pl.BlockSpec(memory_space=pltpu.MemorySpace.VMEM)
    return pl.pallas_call(
        _m3_body,
        in_specs=[vmem, hbm, hbm],
        out_specs=hbm,
        out_shape=jax.ShapeDtypeStruct((DE * E,), jnp.float32),
        scratch_shapes=[
            pltpu.VMEM((DE, _BC), jnp.float32),
            pltpu.VMEM((DE, _BC), jnp.float32),
            pltpu.VMEM((DE, _BC), jnp.float32),
            pltpu.VMEM((DE, _BC), jnp.float32),
            pltpu.VMEM((_BC,), jnp.int32),
            pltpu.VMEM((_BC,), jnp.int32),
            pltpu.SemaphoreType.DMA,
            pltpu.SemaphoreType.DMA,
            pltpu.SemaphoreType.DMA,
            pltpu.SemaphoreType.DMA,
        ],
    )(W_ee, edges_flat, s2)


def kernel(nodes, edges, receivers, senders, W_node, W_edge):
    Wn_n = W_node[:, :DN]
    Wn_sr = W_node[:, DN:DN + 2 * DE]
    W_ee = W_edge[:, :DE]
    W_sr = jnp.concatenate([W_edge[:, DE:DE + DN], W_edge[:, DE + DN:]], axis=0)
    counts = _count_kernel(senders, receivers)
    sr2 = _rewrite_kernel(senders, receivers, counts)

    edges_flat = edges.reshape(-1)
    for _ in range(2):
        C2_flat = _m3(edges_flat, sr2, W_ee)
        SR_flat = _segsum_kernel(edges_flat, senders, receivers)
        nodes, AB_flat = _m1(nodes, SR_flat, Wn_n, Wn_sr, W_sr)
        edges_flat = _edge_kernel(C2_flat, AB_flat, sr2)
    return nodes, edges_flat.reshape(DE, E), receivers, senders
